# baseline (speedup 1.0000x reference)
import functools

import jax
import jax.numpy as jnp
from jax import lax
from jax.experimental import pallas as pl
from jax.experimental.pallas import tpu as pltpu


def _e_kernel(tok_ref, emb_ref, h_ref, wih_ref, whh_ref, bih_ref, bhh_ref,
              hnew_ref, x_ref, x3, sem):
    nb = x3.shape[0]
    base = pl.program_id(0) * nb
    for i in range(nb):
        pltpu.make_async_copy(emb_ref.at[tok_ref[base + i]], x3.at[i], sem).start()
    for i in range(nb):
        pltpu.make_async_copy(emb_ref.at[tok_ref[base + i]], x3.at[i], sem).wait()
    x = x3[...].reshape(nb, x3.shape[2])
    x_ref[...] = x
    h = h_ref[...]
    gi = lax.dot_general(x, wih_ref[...], (((1,), (1,)), ((), ())),
                         preferred_element_type=jnp.float32) + bih_ref[...]
    gh = lax.dot_general(h, whh_ref[...], (((1,), (1,)), ((), ())),
                         preferred_element_type=jnp.float32) + bhh_ref[...]
    hh = h.shape[1]
    r = jax.nn.sigmoid(gi[:, :hh] + gh[:, :hh])
    z = jax.nn.sigmoid(gi[:, hh:2 * hh] + gh[:, hh:2 * hh])
    n = jnp.tanh(gi[:, 2 * hh:] + r * gh[:, 2 * hh:])
    hnew_ref[...] = (1.0 - z) * n + z * h


def _a_kernel(enc_ref, h_ref, x_ref, whT_ref, wsT_ref, vT_ref, attb_ref,
              gwh_ref, gwc_ref, gwx_ref, genb_ref, ohwh_ref, ohwc_ref, ohb_ref,
              o_ref, pgen_ref, ad_ref, scT_s, ctx_s):
    bn = h_ref.shape[0]
    h = h_ref[...]
    hT = jnp.transpose(h)
    hsT = wsT_ref[...] * hT + attb_ref[0, 0]
    whT = whT_ref[...]
    vT = vT_ref[...]
    for i in range(bn):
        encb = enc_ref[i]
        eT = jnp.transpose(encb)
        a = jnp.tanh(eT * whT + hsT[:, i:i + 1])
        scT_s[:, i:i + 1] = jnp.sum(a * vT, axis=0, keepdims=True).T
    scores = jnp.transpose(scT_s[...])
    m = jnp.max(scores, axis=1, keepdims=True)
    e = jnp.exp(scores - m)
    ad = e * (1.0 / jnp.sum(e, axis=1, keepdims=True))
    ad_ref[...] = ad
    adT = jnp.transpose(ad)
    for i in range(bn):
        encb = enc_ref[i]
        ctx_s[i:i + 1, :] = jnp.sum(encb * adT[:, i:i + 1], axis=0, keepdims=True)
    ctx = ctx_s[...]
    o = (lax.dot_general(h, ohwh_ref[...], (((1,), (1,)), ((), ())),
                         preferred_element_type=jnp.float32)
         + lax.dot_general(ctx, ohwc_ref[...], (((1,), (1,)), ((), ())),
                           preferred_element_type=jnp.float32)
         + ohb_ref[...])
    o_ref[...] = o
    g = (lax.dot_general(h, gwh_ref[...], (((1,), (1,)), ((), ())),
                         preferred_element_type=jnp.float32)
         + lax.dot_general(ctx, gwc_ref[...], (((1,), (1,)), ((), ())),
                           preferred_element_type=jnp.float32)
         + lax.dot_general(x_ref[...], gwx_ref[...], (((1,), (1,)), ((), ())),
                           preferred_element_type=jnp.float32)
         + genb_ref[0, 0])
    pgen_ref[...] = jnp.broadcast_to(jax.nn.sigmoid(g), pgen_ref.shape)


def _c_kernel(fiv_ref, val_ref, tid_ref, *, s_len, bn):
    tid_ref[...] = jnp.zeros_like(tid_ref)
    base = pl.program_id(0) * (bn * s_len)
    iota = lax.broadcasted_iota(jnp.int32, (1, 128), 1)

    def body(it, carry):
        s0 = it * 8
        for ds in range(8):
            g0 = base + s0 + ds
            updates = []
            for j in range(bn):
                p = fiv_ref[g0 + j * s_len]
                v = val_ref[g0 + j * s_len]
                off = pl.multiple_of((p >> 7) << 7, 128)
                lane = p & 127
                cur = tid_ref[j:j + 1, pl.ds(off, 128)]
                contrib = jnp.where(iota == lane, v, 0.0)
                updates.append((j, off, cur + contrib))
            for (j, off, nv) in updates:
                tid_ref[j:j + 1, pl.ds(off, 128)] = nv
        return carry

    lax.fori_loop(0, s_len // 8, body, 0)


def _b1_kernel(o_ref, w_ref, b_ref, mc_ref, sc_ref, *, v_total, chunk):
    cg = pl.program_id(0)
    logits = lax.dot_general(o_ref[...], w_ref[...], (((1,), (1,)), ((), ())),
                             preferred_element_type=jnp.float32) + b_ref[...]
    jcol = lax.broadcasted_iota(jnp.int32, logits.shape, 1)
    logits = jnp.where(jcol < (v_total - cg * chunk), logits, -1e30)
    m = jnp.max(logits, axis=1, keepdims=True)
    s = jnp.sum(jnp.exp(logits - m), axis=1, keepdims=True)
    mc_ref[0] = m
    sc_ref[0] = s


def _b2_kernel(o_ref, w_ref, b_ref, mc_ref, sc_ref, pg_ref, tid_ref,
               pv_ref, pf_ref, *, v_total, chunk, nchunks):
    cg = pl.program_id(0)
    mall = mc_ref[...]
    m = jnp.max(mall, axis=0)
    sall = sc_ref[...]
    s = jnp.sum(sall * jnp.exp(mall - m[None]), axis=0)
    rinv = 1.0 / s
    logits = lax.dot_general(o_ref[...], w_ref[...], (((1,), (1,)), ((), ())),
                             preferred_element_type=jnp.float32) + b_ref[...]
    jcol = lax.broadcasted_iota(jnp.int32, logits.shape, 1)
    logits = jnp.where(jcol < (v_total - cg * chunk), logits, -1e30)
    pv = jnp.exp(logits - m) * rinv
    pv_ref[...] = pv
    pg = pg_ref[:, 0:1]
    pf_ref[...] = pv * pg + (1.0 - pg) * tid_ref[...]


def kernel(input_token, last_decoder_hidden, encoder_states, full_input_var,
           emb_table, gru_w_ih, gru_w_hh, gru_b_ih, gru_b_hh,
           w_h, w_s, att_bias, attn_v, gen_w, gen_b,
           outh_w, outh_b, outv_w, outv_b):
    B, S, H = encoder_states.shape
    V, E = emb_table.shape
    PAD = 250
    VE = V + PAD
    NR = (VE + 127) // 128
    VEP = NR * 128
    CHUNK = 2048
    NC = (VE + CHUNK - 1) // CHUNK
    BN = 8
    EB = B // 2

    f32 = jnp.float32
    tok = input_token.reshape(B).astype(jnp.int32)
    emb3 = emb_table.reshape(V, 1, E)

    h_new, x = pl.pallas_call(
        _e_kernel,
        grid=(2,),
        in_specs=[
            pl.BlockSpec(memory_space=pltpu.SMEM),
            pl.BlockSpec(memory_space=pl.ANY),
            pl.BlockSpec((EB, H), lambda p: (p, 0)),
            pl.BlockSpec((3 * H, E), lambda p: (0, 0)),
            pl.BlockSpec((3 * H, H), lambda p: (0, 0)),
            pl.BlockSpec((1, 3 * H), lambda p: (0, 0)),
            pl.BlockSpec((1, 3 * H), lambda p: (0, 0)),
        ],
        out_specs=[
            pl.BlockSpec((EB, H), lambda p: (p, 0)),
            pl.BlockSpec((EB, E), lambda p: (p, 0)),
        ],
        out_shape=[
            jax.ShapeDtypeStruct((B, H), f32),
            jax.ShapeDtypeStruct((B, E), f32),
        ],
        scratch_shapes=[
            pltpu.VMEM((EB, 1, E), f32),
            pltpu.SemaphoreType.DMA,
        ],
        compiler_params=pltpu.CompilerParams(
            dimension_semantics=("parallel",)),
        name="embed_gru",
    )(tok, emb3, last_decoder_hidden, gru_w_ih, gru_w_hh,
      gru_b_ih.reshape(1, 3 * H), gru_b_hh.reshape(1, 3 * H))

    o, pgen_b, att_dist = pl.pallas_call(
        _a_kernel,
        grid=(B // BN,),
        in_specs=[
            pl.BlockSpec((BN, S, H), lambda i: (i, 0, 0)),
            pl.BlockSpec((BN, H), lambda i: (i, 0)),
            pl.BlockSpec((BN, E), lambda i: (i, 0)),
            pl.BlockSpec((H, 1), lambda i: (0, 0)),
            pl.BlockSpec((H, 1), lambda i: (0, 0)),
            pl.BlockSpec((H, 1), lambda i: (0, 0)),
            pl.BlockSpec(memory_space=pltpu.SMEM),
            pl.BlockSpec((1, H), lambda i: (0, 0)),
            pl.BlockSpec((1, H), lambda i: (0, 0)),
            pl.BlockSpec((1, E), lambda i: (0, 0)),
            pl.BlockSpec(memory_space=pltpu.SMEM),
            pl.BlockSpec((E, H), lambda i: (0, 0)),
            pl.BlockSpec((E, H), lambda i: (0, 0)),
            pl.BlockSpec((1, E), lambda i: (0, 0)),
        ],
        out_specs=[
            pl.BlockSpec((BN, E), lambda i: (i, 0)),
            pl.BlockSpec((BN, 128), lambda i: (i, 0)),
            pl.BlockSpec((BN, S), lambda i: (i, 0)),
        ],
        out_shape=[
            jax.ShapeDtypeStruct((B, E), f32),
            jax.ShapeDtypeStruct((B, 128), f32),
            jax.ShapeDtypeStruct((B, S), f32),
        ],
        scratch_shapes=[
            pltpu.VMEM((S, BN), f32),
            pltpu.VMEM((BN, H), f32),
        ],
        compiler_params=pltpu.CompilerParams(
            dimension_semantics=("parallel",)),
        name="attn_pgen",
    )(encoder_states, h_new, x,
      w_h.reshape(H, 1), w_s.reshape(H, 1), attn_v.reshape(H, 1),
      att_bias.reshape(1, 1),
      gen_w[:, :H], gen_w[:, H:2 * H], gen_w[:, 2 * H:],
      gen_b.reshape(1, 1),
      outh_w[:, :H], outh_w[:, H:], outh_b.reshape(1, E))

    tid = pl.pallas_call(
        functools.partial(_c_kernel, s_len=S, bn=BN),
        grid=(B // BN,),
        in_specs=[
            pl.BlockSpec(memory_space=pltpu.SMEM),
            pl.BlockSpec(memory_space=pltpu.SMEM),
        ],
        out_specs=pl.BlockSpec((BN, VEP), lambda i: (i, 0)),
        out_shape=jax.ShapeDtypeStruct((B, VEP), f32),
        compiler_params=pltpu.CompilerParams(
            dimension_semantics=("parallel",)),
        name="ptr_scatter",
    )(full_input_var.reshape(B * S).astype(jnp.int32),
      att_dist.reshape(B * S))

    mC, sC = pl.pallas_call(
        functools.partial(_b1_kernel, v_total=V, chunk=CHUNK),
        grid=(NC,),
        in_specs=[
            pl.BlockSpec((B, E), lambda c: (0, 0)),
            pl.BlockSpec((CHUNK, E), lambda c: (c, 0)),
            pl.BlockSpec((1, CHUNK), lambda c: (0, c)),
        ],
        out_specs=[
            pl.BlockSpec((1, B, 1), lambda c: (c, 0, 0)),
            pl.BlockSpec((1, B, 1), lambda c: (c, 0, 0)),
        ],
        out_shape=[
            jax.ShapeDtypeStruct((NC, B, 1), f32),
            jax.ShapeDtypeStruct((NC, B, 1), f32),
        ],
        compiler_params=pltpu.CompilerParams(
            dimension_semantics=("parallel",)),
        name="logit_stats",
    )(o, outv_w, outv_b.reshape(1, V))

    p_vocab, p_final = pl.pallas_call(
        functools.partial(_b2_kernel, v_total=V, chunk=CHUNK, nchunks=NC),
        grid=(NC,),
        in_specs=[
            pl.BlockSpec((B, E), lambda c: (0, 0)),
            pl.BlockSpec((CHUNK, E), lambda c: (c, 0)),
            pl.BlockSpec((1, CHUNK), lambda c: (0, c)),
            pl.BlockSpec((NC, B, 1), lambda c: (0, 0, 0)),
            pl.BlockSpec((NC, B, 1), lambda c: (0, 0, 0)),
            pl.BlockSpec((B, 128), lambda c: (0, 0)),
            pl.BlockSpec((B, CHUNK), lambda c: (0, c)),
        ],
        out_specs=[
            pl.BlockSpec((B, CHUNK), lambda c: (0, c)),
            pl.BlockSpec((B, CHUNK), lambda c: (0, c)),
        ],
        out_shape=[
            jax.ShapeDtypeStruct((B, V), f32),
            jax.ShapeDtypeStruct((B, VE), f32),
        ],
        compiler_params=pltpu.CompilerParams(
            dimension_semantics=("parallel",)),
        name="vocab_mix",
    )(o, outv_w, outv_b.reshape(1, V), mC, sC, pgen_b, tid)

    p_gen = pgen_b[:, 0:1]
    return (h_new, p_final, p_gen, p_vocab, att_dist)


# speedup vs baseline: 1.2696x; 1.2696x over previous
import functools

import jax
import jax.numpy as jnp
from jax import lax
from jax.experimental import pallas as pl
from jax.experimental.pallas import tpu as pltpu

_SEM = ("parallel", "arbitrary")


def _e_kernel(tok_ref, emb_ref, h_ref, wih_ref, whh_ref, bih_ref, bhh_ref,
              hnew_ref, x_ref, x3, sem):
    nb = x3.shape[0]
    base = pl.program_id(0) * nb
    for i in range(nb):
        pltpu.make_async_copy(emb_ref.at[tok_ref[base + i]], x3.at[i], sem).start()
    for i in range(nb):
        pltpu.make_async_copy(emb_ref.at[tok_ref[base + i]], x3.at[i], sem).wait()
    x = x3[...].reshape(nb, x3.shape[2])
    x_ref[...] = x
    h = h_ref[...]
    gi = lax.dot_general(x, wih_ref[...], (((1,), (1,)), ((), ())),
                         preferred_element_type=jnp.float32) + bih_ref[...]
    gh = lax.dot_general(h, whh_ref[...], (((1,), (1,)), ((), ())),
                         preferred_element_type=jnp.float32) + bhh_ref[...]
    hh = h.shape[1]
    r = jax.nn.sigmoid(gi[:, :hh] + gh[:, :hh])
    z = jax.nn.sigmoid(gi[:, hh:2 * hh] + gh[:, hh:2 * hh])
    n = jnp.tanh(gi[:, 2 * hh:] + r * gh[:, 2 * hh:])
    hnew_ref[...] = (1.0 - z) * n + z * h


def _a_kernel(enc_ref, h_ref, x_ref, whT_ref, wsT_ref, vT_ref, attb_ref,
              gwh_ref, gwc_ref, gwx_ref, genb_ref, ohwh_ref, ohwc_ref, ohb_ref,
              o_ref, pgen_ref, ad_ref, scT_s, ctx_s):
    bn = h_ref.shape[0]
    h = h_ref[...]
    hT = jnp.transpose(h)
    hsT = wsT_ref[...] * hT + attb_ref[0, 0]
    whT = whT_ref[...]
    vT = vT_ref[...]
    for i in range(bn):
        encb = enc_ref[i]
        eT = jnp.transpose(encb)
        a = jnp.tanh(eT * whT + hsT[:, i:i + 1])
        scT_s[:, i:i + 1] = jnp.sum(a * vT, axis=0, keepdims=True).T
    scores = jnp.transpose(scT_s[...])
    m = jnp.max(scores, axis=1, keepdims=True)
    e = jnp.exp(scores - m)
    ad = e * (1.0 / jnp.sum(e, axis=1, keepdims=True))
    ad_ref[...] = ad
    adT = jnp.transpose(ad)
    for i in range(bn):
        encb = enc_ref[i]
        ctx_s[i:i + 1, :] = jnp.sum(encb * adT[:, i:i + 1], axis=0, keepdims=True)
    ctx = ctx_s[...]
    o = (lax.dot_general(h, ohwh_ref[...], (((1,), (1,)), ((), ())),
                         preferred_element_type=jnp.float32)
         + lax.dot_general(ctx, ohwc_ref[...], (((1,), (1,)), ((), ())),
                           preferred_element_type=jnp.float32)
         + ohb_ref[...])
    o_ref[...] = o
    g = (lax.dot_general(h, gwh_ref[...], (((1,), (1,)), ((), ())),
                         preferred_element_type=jnp.float32)
         + lax.dot_general(ctx, gwc_ref[...], (((1,), (1,)), ((), ())),
                           preferred_element_type=jnp.float32)
         + lax.dot_general(x_ref[...], gwx_ref[...], (((1,), (1,)), ((), ())),
                           preferred_element_type=jnp.float32)
         + genb_ref[0, 0])
    pgen_ref[...] = jnp.broadcast_to(jax.nn.sigmoid(g), pgen_ref.shape)


def _c_kernel(q_ref, l_ref, ad_ref, tid_ref, *, s_len, bn, nrp):
    qT = jnp.transpose(q_ref[...])
    lT = jnp.transpose(l_ref[...])
    aT = jnp.transpose(ad_ref[...])
    iq = lax.broadcasted_iota(jnp.int32, (1, nrp), 1)
    il = lax.broadcasted_iota(jnp.int32, (1, 128), 1)
    for j in range(bn):
        pmat = jnp.where(qT[:, j:j + 1] == iq, aT[:, j:j + 1], 0.0)
        momat = jnp.where(lT[:, j:j + 1] == il, 1.0, 0.0)
        tb = lax.dot_general(pmat, momat, (((0,), (0,)), ((), ())),
                             preferred_element_type=jnp.float32)
        tid_ref[j * nrp:(j + 1) * nrp, :] = tb


def _b1_kernel(o_ref, w_ref, b_ref, mc_ref, sc_ref, *, v_total, chunk, nck, nc):
    cg = jnp.minimum(pl.program_id(0) * nck + pl.program_id(1), nc - 1)
    logits = lax.dot_general(o_ref[...], w_ref[...], (((1,), (1,)), ((), ())),
                             preferred_element_type=jnp.float32) + b_ref[...]
    jcol = lax.broadcasted_iota(jnp.int32, logits.shape, 1)
    logits = jnp.where(jcol < (v_total - cg * chunk), logits, -1e30)
    m = jnp.max(logits, axis=1, keepdims=True)
    s = jnp.sum(jnp.exp(logits - m), axis=1, keepdims=True)
    mc_ref[0] = m
    sc_ref[0] = s


def _b2_kernel(o_ref, w_ref, b_ref, mc_ref, sc_ref, pg_ref, tid_ref,
               pv_ref, pf_ref, *, v_total, chunk, nck, nc):
    cg = jnp.minimum(pl.program_id(0) * nck + pl.program_id(1), nc - 1)
    mall = mc_ref[...]
    m = jnp.max(mall, axis=0)
    sall = sc_ref[...]
    s = jnp.sum(sall * jnp.exp(mall - m[None]), axis=0)
    rinv = 1.0 / s
    logits = lax.dot_general(o_ref[...], w_ref[...], (((1,), (1,)), ((), ())),
                             preferred_element_type=jnp.float32) + b_ref[...]
    jcol = lax.broadcasted_iota(jnp.int32, logits.shape, 1)
    logits = jnp.where(jcol < (v_total - cg * chunk), logits, -1e30)
    pv = jnp.exp(logits - m) * rinv
    pv_ref[...] = pv
    pg = pg_ref[:, 0:1]
    pf_ref[...] = pv * pg + (1.0 - pg) * tid_ref[...]


def kernel(input_token, last_decoder_hidden, encoder_states, full_input_var,
           emb_table, gru_w_ih, gru_w_hh, gru_b_ih, gru_b_hh,
           w_h, w_s, att_bias, attn_v, gen_w, gen_b,
           outh_w, outh_b, outv_w, outv_b):
    B, S, H = encoder_states.shape
    V, E = emb_table.shape
    PAD = 250
    VE = V + PAD
    NR = (VE + 127) // 128
    VEP = NR * 128
    CHUNK = 2048
    NC = 25
    NCK = 13
    BN = 8
    NBLK = (B // BN) // 2
    EB = B // 2

    f32 = jnp.float32
    cgc = lambda p, c: jnp.minimum(p * NCK + c, NC - 1)
    tok = input_token.reshape(B).astype(jnp.int32)
    emb3 = emb_table.reshape(V, 1, E)

    h_new, x = pl.pallas_call(
        _e_kernel,
        grid=(2, 1),
        in_specs=[
            pl.BlockSpec(memory_space=pltpu.SMEM),
            pl.BlockSpec(memory_space=pl.ANY),
            pl.BlockSpec((EB, H), lambda p, q: (p, 0)),
            pl.BlockSpec((3 * H, E), lambda p, q: (0, 0)),
            pl.BlockSpec((3 * H, H), lambda p, q: (0, 0)),
            pl.BlockSpec((1, 3 * H), lambda p, q: (0, 0)),
            pl.BlockSpec((1, 3 * H), lambda p, q: (0, 0)),
        ],
        out_specs=[
            pl.BlockSpec((EB, H), lambda p, q: (p, 0)),
            pl.BlockSpec((EB, E), lambda p, q: (p, 0)),
        ],
        out_shape=[
            jax.ShapeDtypeStruct((B, H), f32),
            jax.ShapeDtypeStruct((B, E), f32),
        ],
        scratch_shapes=[
            pltpu.VMEM((EB, 1, E), f32),
            pltpu.SemaphoreType.DMA,
        ],
        compiler_params=pltpu.CompilerParams(dimension_semantics=_SEM),
        name="embed_gru",
    )(tok, emb3, last_decoder_hidden, gru_w_ih, gru_w_hh,
      gru_b_ih.reshape(1, 3 * H), gru_b_hh.reshape(1, 3 * H))

    o, pgen_b, att_dist = pl.pallas_call(
        _a_kernel,
        grid=(2, NBLK),
        in_specs=[
            pl.BlockSpec((BN, S, H), lambda p, i: (p * NBLK + i, 0, 0)),
            pl.BlockSpec((BN, H), lambda p, i: (p * NBLK + i, 0)),
            pl.BlockSpec((BN, E), lambda p, i: (p * NBLK + i, 0)),
            pl.BlockSpec((H, 1), lambda p, i: (0, 0)),
            pl.BlockSpec((H, 1), lambda p, i: (0, 0)),
            pl.BlockSpec((H, 1), lambda p, i: (0, 0)),
            pl.BlockSpec(memory_space=pltpu.SMEM),
            pl.BlockSpec((1, H), lambda p, i: (0, 0)),
            pl.BlockSpec((1, H), lambda p, i: (0, 0)),
            pl.BlockSpec((1, E), lambda p, i: (0, 0)),
            pl.BlockSpec(memory_space=pltpu.SMEM),
            pl.BlockSpec((E, H), lambda p, i: (0, 0)),
            pl.BlockSpec((E, H), lambda p, i: (0, 0)),
            pl.BlockSpec((1, E), lambda p, i: (0, 0)),
        ],
        out_specs=[
            pl.BlockSpec((BN, E), lambda p, i: (p * NBLK + i, 0)),
            pl.BlockSpec((BN, 128), lambda p, i: (p * NBLK + i, 0)),
            pl.BlockSpec((BN, S), lambda p, i: (p * NBLK + i, 0)),
        ],
        out_shape=[
            jax.ShapeDtypeStruct((B, E), f32),
            jax.ShapeDtypeStruct((B, 128), f32),
            jax.ShapeDtypeStruct((B, S), f32),
        ],
        scratch_shapes=[
            pltpu.VMEM((S, BN), f32),
            pltpu.VMEM((BN, H), f32),
        ],
        compiler_params=pltpu.CompilerParams(dimension_semantics=_SEM),
        name="attn_pgen",
    )(encoder_states, h_new, x,
      w_h.reshape(H, 1), w_s.reshape(H, 1), attn_v.reshape(H, 1),
      att_bias.reshape(1, 1),
      gen_w[:, :H], gen_w[:, H:2 * H], gen_w[:, 2 * H:],
      gen_b.reshape(1, 1),
      outh_w[:, :H], outh_w[:, H:], outh_b.reshape(1, E))

    NRP = 400
    fiv = full_input_var.astype(jnp.int32)
    tid = pl.pallas_call(
        functools.partial(_c_kernel, s_len=S, bn=BN, nrp=NRP),
        grid=(2, NBLK),
        in_specs=[
            pl.BlockSpec((BN, S), lambda p, i: (p * NBLK + i, 0)),
            pl.BlockSpec((BN, S), lambda p, i: (p * NBLK + i, 0)),
            pl.BlockSpec((BN, S), lambda p, i: (p * NBLK + i, 0)),
        ],
        out_specs=pl.BlockSpec((BN * NRP, 128), lambda p, i: (p * NBLK + i, 0)),
        out_shape=jax.ShapeDtypeStruct((B * NRP, 128), f32),
        compiler_params=pltpu.CompilerParams(dimension_semantics=_SEM),
        name="ptr_scatter",
    )(fiv >> 7, fiv & 127, att_dist)
    tid2 = tid.reshape(B, NRP * 128)

    mC, sC = pl.pallas_call(
        functools.partial(_b1_kernel, v_total=V, chunk=CHUNK, nck=NCK, nc=NC),
        grid=(2, NCK),
        in_specs=[
            pl.BlockSpec((B, E), lambda p, c: (0, 0)),
            pl.BlockSpec((CHUNK, E), lambda p, c: (cgc(p, c), 0)),
            pl.BlockSpec((1, CHUNK), lambda p, c: (0, cgc(p, c))),
        ],
        out_specs=[
            pl.BlockSpec((1, B, 1), lambda p, c: (cgc(p, c), 0, 0)),
            pl.BlockSpec((1, B, 1), lambda p, c: (cgc(p, c), 0, 0)),
        ],
        out_shape=[
            jax.ShapeDtypeStruct((NC, B, 1), f32),
            jax.ShapeDtypeStruct((NC, B, 1), f32),
        ],
        compiler_params=pltpu.CompilerParams(dimension_semantics=_SEM),
        name="logit_stats",
    )(o, outv_w, outv_b.reshape(1, V))

    p_vocab, p_final = pl.pallas_call(
        functools.partial(_b2_kernel, v_total=V, chunk=CHUNK, nck=NCK, nc=NC),
        grid=(2, NCK),
        in_specs=[
            pl.BlockSpec((B, E), lambda p, c: (0, 0)),
            pl.BlockSpec((CHUNK, E), lambda p, c: (cgc(p, c), 0)),
            pl.BlockSpec((1, CHUNK), lambda p, c: (0, cgc(p, c))),
            pl.BlockSpec((NC, B, 1), lambda p, c: (0, 0, 0)),
            pl.BlockSpec((NC, B, 1), lambda p, c: (0, 0, 0)),
            pl.BlockSpec((B, 128), lambda p, c: (0, 0)),
            pl.BlockSpec((B, CHUNK), lambda p, c: (0, cgc(p, c))),
        ],
        out_specs=[
            pl.BlockSpec((B, CHUNK), lambda p, c: (0, cgc(p, c))),
            pl.BlockSpec((B, CHUNK), lambda p, c: (0, cgc(p, c))),
        ],
        out_shape=[
            jax.ShapeDtypeStruct((B, V), f32),
            jax.ShapeDtypeStruct((B, VE), f32),
        ],
        compiler_params=pltpu.CompilerParams(dimension_semantics=_SEM),
        name="vocab_mix",
    )(o, outv_w, outv_b.reshape(1, V), mC, sC, pgen_b, tid2)

    p_gen = pgen_b[:, 0:1]
    return (h_new, p_final, p_gen, p_vocab, att_dist)


# speedup vs baseline: 1.5109x; 1.1901x over previous
import functools

import jax
import jax.numpy as jnp
from jax import lax
from jax.experimental import pallas as pl
from jax.experimental.pallas import tpu as pltpu

_SEM = ("parallel", "arbitrary")


def _e_kernel(tok_ref, emb_ref, h_ref, wih_ref, whh_ref, bih_ref, bhh_ref,
              hnew_ref, x_ref, x3, sem):
    nb = x3.shape[0]
    base = pl.program_id(0) * nb
    for i in range(nb):
        pltpu.make_async_copy(emb_ref.at[tok_ref[base + i]], x3.at[i], sem).start()
    for i in range(nb):
        pltpu.make_async_copy(emb_ref.at[tok_ref[base + i]], x3.at[i], sem).wait()
    x = x3[...].reshape(nb, x3.shape[2])
    x_ref[...] = x
    h = h_ref[...]
    gi = lax.dot_general(x, wih_ref[...], (((1,), (1,)), ((), ())),
                         preferred_element_type=jnp.float32) + bih_ref[...]
    gh = lax.dot_general(h, whh_ref[...], (((1,), (1,)), ((), ())),
                         preferred_element_type=jnp.float32) + bhh_ref[...]
    hh = h.shape[1]
    r = jax.nn.sigmoid(gi[:, :hh] + gh[:, :hh])
    z = jax.nn.sigmoid(gi[:, hh:2 * hh] + gh[:, hh:2 * hh])
    n = jnp.tanh(gi[:, 2 * hh:] + r * gh[:, 2 * hh:])
    hnew_ref[...] = (1.0 - z) * n + z * h


def _a_kernel(enc_ref, h_ref, x_ref, wh_ref, ws_ref, vT_ref, attb_ref,
              gwh_ref, gwc_ref, gwx_ref, genb_ref, ohwh_ref, ohwc_ref, ohb_ref,
              o_ref, pgen_ref, ad_ref, scT_s, ctx_s):
    bn = h_ref.shape[0]
    h = h_ref[...]
    hsum = ws_ref[...] * h + attb_ref[0, 0]
    wh = wh_ref[...]
    vT = vT_ref[...]
    for i in range(bn):
        encb = enc_ref[i]
        a = jnp.tanh(encb * wh + hsum[i:i + 1, :])
        scT_s[:, i:i + 1] = lax.dot_general(
            a, vT, (((1,), (0,)), ((), ())),
            preferred_element_type=jnp.float32)
    scores = jnp.transpose(scT_s[...])
    m = jnp.max(scores, axis=1, keepdims=True)
    e = jnp.exp(scores - m)
    ad = e * (1.0 / jnp.sum(e, axis=1, keepdims=True))
    ad_ref[...] = ad
    adT = jnp.transpose(ad)
    for i in range(bn):
        encb = enc_ref[i]
        ctx_s[i:i + 1, :] = jnp.sum(encb * adT[:, i:i + 1], axis=0, keepdims=True)
    ctx = ctx_s[...]
    o = (lax.dot_general(h, ohwh_ref[...], (((1,), (1,)), ((), ())),
                         preferred_element_type=jnp.float32)
         + lax.dot_general(ctx, ohwc_ref[...], (((1,), (1,)), ((), ())),
                           preferred_element_type=jnp.float32)
         + ohb_ref[...])
    o_ref[...] = o
    g = (lax.dot_general(h, gwh_ref[...], (((1,), (1,)), ((), ())),
                         preferred_element_type=jnp.float32)
         + lax.dot_general(ctx, gwc_ref[...], (((1,), (1,)), ((), ())),
                           preferred_element_type=jnp.float32)
         + lax.dot_general(x_ref[...], gwx_ref[...], (((1,), (1,)), ((), ())),
                           preferred_element_type=jnp.float32)
         + genb_ref[0, 0])
    pgen_ref[...] = jnp.broadcast_to(jax.nn.sigmoid(g), pgen_ref.shape)


def _c_kernel(q_ref, l_ref, ad_ref, tid_ref, *, s_len, bn, nrp):
    qT = jnp.transpose(q_ref[...])
    lT = jnp.transpose(l_ref[...])
    aT = jnp.transpose(ad_ref[...])
    iq = lax.broadcasted_iota(jnp.int32, (1, nrp), 1)
    il = lax.broadcasted_iota(jnp.int32, (1, 128), 1)
    for j in range(bn):
        pmat = jnp.where(qT[:, j:j + 1] == iq, aT[:, j:j + 1], 0.0)
        momat = jnp.where(lT[:, j:j + 1] == il, 1.0, 0.0)
        tb = lax.dot_general(pmat, momat, (((0,), (0,)), ((), ())),
                             preferred_element_type=jnp.float32)
        tid_ref[j * nrp:(j + 1) * nrp, :] = tb


def _b1_kernel(o_ref, w_ref, b_ref, mc_ref, sc_ref, *, v_total, chunk, nck, nc):
    cg = jnp.minimum(pl.program_id(0) * nck + pl.program_id(1), nc - 1)
    logits = lax.dot_general(o_ref[...], w_ref[...], (((1,), (1,)), ((), ())),
                             preferred_element_type=jnp.float32) + b_ref[...]
    jcol = lax.broadcasted_iota(jnp.int32, logits.shape, 1)
    logits = jnp.where(jcol < (v_total - cg * chunk), logits, -1e30)
    m = jnp.max(logits, axis=1, keepdims=True)
    s = jnp.sum(jnp.exp(logits - m), axis=1, keepdims=True)
    mc_ref[0] = m
    sc_ref[0] = s


def _b2_kernel(o_ref, w_ref, b_ref, mc_ref, sc_ref, pg_ref, tid_ref,
               pv_ref, pf_ref, m_sc, r_sc, *, v_total, chunk, nck, nc):
    cg = jnp.minimum(pl.program_id(0) * nck + pl.program_id(1), nc - 1)

    @pl.when(pl.program_id(1) == 0)
    def _():
        mall = mc_ref[...]
        mm = jnp.max(mall, axis=0)
        ss = jnp.sum(sc_ref[...] * jnp.exp(mall - mm[None]), axis=0)
        m_sc[...] = mm
        r_sc[...] = 1.0 / ss

    m = m_sc[...]
    rinv = r_sc[...]
    logits = lax.dot_general(o_ref[...], w_ref[...], (((1,), (1,)), ((), ())),
                             preferred_element_type=jnp.float32) + b_ref[...]
    jcol = lax.broadcasted_iota(jnp.int32, logits.shape, 1)
    logits = jnp.where(jcol < (v_total - cg * chunk), logits, -1e30)
    pv = jnp.exp(logits - m) * rinv
    pv_ref[...] = pv
    pg = pg_ref[:, 0:1]
    pf_ref[...] = pv * pg + (1.0 - pg) * tid_ref[...]


def kernel(input_token, last_decoder_hidden, encoder_states, full_input_var,
           emb_table, gru_w_ih, gru_w_hh, gru_b_ih, gru_b_hh,
           w_h, w_s, att_bias, attn_v, gen_w, gen_b,
           outh_w, outh_b, outv_w, outv_b):
    B, S, H = encoder_states.shape
    V, E = emb_table.shape
    PAD = 250
    VE = V + PAD
    NR = (VE + 127) // 128
    VEP = NR * 128
    CHUNK = 2048
    NC = 25
    NCK = 13
    BN = 8
    NBLK = (B // BN) // 2
    EB = B // 2

    f32 = jnp.float32
    cgc = lambda p, c: jnp.minimum(p * NCK + c, NC - 1)
    tok = input_token.reshape(B).astype(jnp.int32)
    emb3 = emb_table.reshape(V, 1, E)

    h_new, x = pl.pallas_call(
        _e_kernel,
        grid=(2, 1),
        in_specs=[
            pl.BlockSpec(memory_space=pltpu.SMEM),
            pl.BlockSpec(memory_space=pl.ANY),
            pl.BlockSpec((EB, H), lambda p, q: (p, 0)),
            pl.BlockSpec((3 * H, E), lambda p, q: (0, 0)),
            pl.BlockSpec((3 * H, H), lambda p, q: (0, 0)),
            pl.BlockSpec((1, 3 * H), lambda p, q: (0, 0)),
            pl.BlockSpec((1, 3 * H), lambda p, q: (0, 0)),
        ],
        out_specs=[
            pl.BlockSpec((EB, H), lambda p, q: (p, 0)),
            pl.BlockSpec((EB, E), lambda p, q: (p, 0)),
        ],
        out_shape=[
            jax.ShapeDtypeStruct((B, H), f32),
            jax.ShapeDtypeStruct((B, E), f32),
        ],
        scratch_shapes=[
            pltpu.VMEM((EB, 1, E), f32),
            pltpu.SemaphoreType.DMA,
        ],
        compiler_params=pltpu.CompilerParams(dimension_semantics=_SEM),
        name="embed_gru",
    )(tok, emb3, last_decoder_hidden, gru_w_ih, gru_w_hh,
      gru_b_ih.reshape(1, 3 * H), gru_b_hh.reshape(1, 3 * H))

    o, pgen_b, att_dist = pl.pallas_call(
        _a_kernel,
        grid=(2, NBLK),
        in_specs=[
            pl.BlockSpec((BN, S, H), lambda p, i: (p * NBLK + i, 0, 0)),
            pl.BlockSpec((BN, H), lambda p, i: (p * NBLK + i, 0)),
            pl.BlockSpec((BN, E), lambda p, i: (p * NBLK + i, 0)),
            pl.BlockSpec((1, H), lambda p, i: (0, 0)),
            pl.BlockSpec((1, H), lambda p, i: (0, 0)),
            pl.BlockSpec((H, 1), lambda p, i: (0, 0)),
            pl.BlockSpec(memory_space=pltpu.SMEM),
            pl.BlockSpec((1, H), lambda p, i: (0, 0)),
            pl.BlockSpec((1, H), lambda p, i: (0, 0)),
            pl.BlockSpec((1, E), lambda p, i: (0, 0)),
            pl.BlockSpec(memory_space=pltpu.SMEM),
            pl.BlockSpec((E, H), lambda p, i: (0, 0)),
            pl.BlockSpec((E, H), lambda p, i: (0, 0)),
            pl.BlockSpec((1, E), lambda p, i: (0, 0)),
        ],
        out_specs=[
            pl.BlockSpec((BN, E), lambda p, i: (p * NBLK + i, 0)),
            pl.BlockSpec((BN, 128), lambda p, i: (p * NBLK + i, 0)),
            pl.BlockSpec((BN, S), lambda p, i: (p * NBLK + i, 0)),
        ],
        out_shape=[
            jax.ShapeDtypeStruct((B, E), f32),
            jax.ShapeDtypeStruct((B, 128), f32),
            jax.ShapeDtypeStruct((B, S), f32),
        ],
        scratch_shapes=[
            pltpu.VMEM((S, BN), f32),
            pltpu.VMEM((BN, H), f32),
        ],
        compiler_params=pltpu.CompilerParams(dimension_semantics=_SEM),
        name="attn_pgen",
    )(encoder_states, h_new, x,
      w_h.reshape(1, H), w_s.reshape(1, H), attn_v.reshape(H, 1),
      att_bias.reshape(1, 1),
      gen_w[:, :H], gen_w[:, H:2 * H], gen_w[:, 2 * H:],
      gen_b.reshape(1, 1),
      outh_w[:, :H], outh_w[:, H:], outh_b.reshape(1, E))

    NRP = 400
    fiv = full_input_var.astype(jnp.int32)
    tid = pl.pallas_call(
        functools.partial(_c_kernel, s_len=S, bn=BN, nrp=NRP),
        grid=(2, NBLK),
        in_specs=[
            pl.BlockSpec((BN, S), lambda p, i: (p * NBLK + i, 0)),
            pl.BlockSpec((BN, S), lambda p, i: (p * NBLK + i, 0)),
            pl.BlockSpec((BN, S), lambda p, i: (p * NBLK + i, 0)),
        ],
        out_specs=pl.BlockSpec((BN * NRP, 128), lambda p, i: (p * NBLK + i, 0)),
        out_shape=jax.ShapeDtypeStruct((B * NRP, 128), f32),
        compiler_params=pltpu.CompilerParams(dimension_semantics=_SEM),
        name="ptr_scatter",
    )(fiv >> 7, fiv & 127, att_dist)
    tid2 = tid.reshape(B, NRP * 128)

    mC, sC = pl.pallas_call(
        functools.partial(_b1_kernel, v_total=V, chunk=CHUNK, nck=NCK, nc=NC),
        grid=(2, NCK),
        in_specs=[
            pl.BlockSpec((B, E), lambda p, c: (0, 0)),
            pl.BlockSpec((CHUNK, E), lambda p, c: (cgc(p, c), 0)),
            pl.BlockSpec((1, CHUNK), lambda p, c: (0, cgc(p, c))),
        ],
        out_specs=[
            pl.BlockSpec((1, B, 1), lambda p, c: (cgc(p, c), 0, 0)),
            pl.BlockSpec((1, B, 1), lambda p, c: (cgc(p, c), 0, 0)),
        ],
        out_shape=[
            jax.ShapeDtypeStruct((NC, B, 1), f32),
            jax.ShapeDtypeStruct((NC, B, 1), f32),
        ],
        compiler_params=pltpu.CompilerParams(dimension_semantics=_SEM),
        name="logit_stats",
    )(o, outv_w, outv_b.reshape(1, V))

    p_vocab, p_final = pl.pallas_call(
        functools.partial(_b2_kernel, v_total=V, chunk=CHUNK, nck=NCK, nc=NC),
        grid=(2, NCK),
        in_specs=[
            pl.BlockSpec((B, E), lambda p, c: (0, 0)),
            pl.BlockSpec((CHUNK, E), lambda p, c: (cgc(p, c), 0)),
            pl.BlockSpec((1, CHUNK), lambda p, c: (0, cgc(p, c))),
            pl.BlockSpec((NC, B, 1), lambda p, c: (0, 0, 0)),
            pl.BlockSpec((NC, B, 1), lambda p, c: (0, 0, 0)),
            pl.BlockSpec((B, 128), lambda p, c: (0, 0)),
            pl.BlockSpec((B, CHUNK), lambda p, c: (0, cgc(p, c))),
        ],
        out_specs=[
            pl.BlockSpec((B, CHUNK), lambda p, c: (0, cgc(p, c))),
            pl.BlockSpec((B, CHUNK), lambda p, c: (0, cgc(p, c))),
        ],
        out_shape=[
            jax.ShapeDtypeStruct((B, V), f32),
            jax.ShapeDtypeStruct((B, VE), f32),
        ],
        scratch_shapes=[
            pltpu.VMEM((B, 1), f32),
            pltpu.VMEM((B, 1), f32),
        ],
        compiler_params=pltpu.CompilerParams(dimension_semantics=_SEM),
        name="vocab_mix",
    )(o, outv_w, outv_b.reshape(1, V), mC, sC, pgen_b, tid2)

    p_gen = pgen_b[:, 0:1]
    return (h_new, p_final, p_gen, p_vocab, att_dist)


# speedup vs baseline: 1.5811x; 1.0465x over previous
import functools

import jax
import jax.numpy as jnp
from jax import lax
from jax.experimental import pallas as pl
from jax.experimental.pallas import tpu as pltpu

_SEM = ("parallel", "arbitrary")


def _e_kernel(tok_ref, emb_ref, h_ref, wih_ref, whh_ref, bih_ref, bhh_ref,
              hnew_ref, x_ref, x3, sem):
    nb = x3.shape[0]
    base = pl.program_id(0) * nb
    for i in range(nb):
        pltpu.make_async_copy(emb_ref.at[tok_ref[base + i]], x3.at[i], sem).start()
    for i in range(nb):
        pltpu.make_async_copy(emb_ref.at[tok_ref[base + i]], x3.at[i], sem).wait()
    x = x3[...].reshape(nb, x3.shape[2])
    x_ref[...] = x
    h = h_ref[...]
    gi = lax.dot_general(x, wih_ref[...], (((1,), (1,)), ((), ())),
                         preferred_element_type=jnp.float32) + bih_ref[...]
    gh = lax.dot_general(h, whh_ref[...], (((1,), (1,)), ((), ())),
                         preferred_element_type=jnp.float32) + bhh_ref[...]
    hh = h.shape[1]
    r = jax.nn.sigmoid(gi[:, :hh] + gh[:, :hh])
    z = jax.nn.sigmoid(gi[:, hh:2 * hh] + gh[:, hh:2 * hh])
    n = jnp.tanh(gi[:, 2 * hh:] + r * gh[:, 2 * hh:])
    hnew_ref[...] = (1.0 - z) * n + z * h


def _a_kernel(enc_ref, h_ref, x_ref, wh_ref, ws_ref, vT_ref, attb_ref,
              gwh_ref, gwc_ref, gwx_ref, genb_ref, ohwh_ref, ohwc_ref, ohb_ref,
              o_ref, pgen_ref, ad_ref, scT_s, ctx_s):
    bn = h_ref.shape[0]
    h = h_ref[...]
    hsum = ws_ref[...] * h + attb_ref[0, 0]
    wh = wh_ref[...]
    vT = vT_ref[...]
    for i in range(bn):
        encb = enc_ref[i]
        a = jnp.tanh(encb * wh + hsum[i:i + 1, :])
        scT_s[:, i:i + 1] = lax.dot_general(
            a, vT, (((1,), (0,)), ((), ())),
            preferred_element_type=jnp.float32)
    scores = jnp.transpose(scT_s[...])
    m = jnp.max(scores, axis=1, keepdims=True)
    e = jnp.exp(scores - m)
    ad = e * (1.0 / jnp.sum(e, axis=1, keepdims=True))
    ad_ref[...] = ad
    adT = jnp.transpose(ad)
    for i in range(bn):
        encb = enc_ref[i]
        ctx_s[i:i + 1, :] = jnp.sum(encb * adT[:, i:i + 1], axis=0, keepdims=True)
    ctx = ctx_s[...]
    o = (lax.dot_general(h, ohwh_ref[...], (((1,), (1,)), ((), ())),
                         preferred_element_type=jnp.float32)
         + lax.dot_general(ctx, ohwc_ref[...], (((1,), (1,)), ((), ())),
                           preferred_element_type=jnp.float32)
         + ohb_ref[...])
    o_ref[...] = o
    g = (lax.dot_general(h, gwh_ref[...], (((1,), (1,)), ((), ())),
                         preferred_element_type=jnp.float32)
         + lax.dot_general(ctx, gwc_ref[...], (((1,), (1,)), ((), ())),
                           preferred_element_type=jnp.float32)
         + lax.dot_general(x_ref[...], gwx_ref[...], (((1,), (1,)), ((), ())),
                           preferred_element_type=jnp.float32)
         + genb_ref[0, 0])
    pgen_ref[...] = jnp.broadcast_to(jax.nn.sigmoid(g), pgen_ref.shape)


def _c_kernel(q_ref, l_ref, ad_ref, tid_ref, *, s_len, bn, nrp):
    qT = jnp.transpose(q_ref[...])
    lT = jnp.transpose(l_ref[...])
    aT = jnp.transpose(ad_ref[...])
    iq = lax.broadcasted_iota(jnp.int32, (1, nrp), 1)
    il = lax.broadcasted_iota(jnp.int32, (1, 128), 1)
    for j in range(bn):
        pmat = jnp.where(qT[:, j:j + 1] == iq, aT[:, j:j + 1],
                         0.0).astype(jnp.bfloat16)
        momat = jnp.where(lT[:, j:j + 1] == il, 1.0, 0.0).astype(jnp.bfloat16)
        tb = lax.dot_general(pmat, momat, (((0,), (0,)), ((), ())),
                             preferred_element_type=jnp.float32)
        tid_ref[j * nrp:(j + 1) * nrp, :] = tb


def _b1_kernel(o_ref, w_ref, b_ref, mc_ref, sc_ref, *, v_total, chunk, nck, nc):
    cg = jnp.minimum(pl.program_id(0) * nck + pl.program_id(1), nc - 1)
    logits = lax.dot_general(o_ref[...], w_ref[...], (((1,), (1,)), ((), ())),
                             preferred_element_type=jnp.float32) + b_ref[...]
    jcol = lax.broadcasted_iota(jnp.int32, logits.shape, 1)
    logits = jnp.where(jcol < (v_total - cg * chunk), logits, -1e30)
    m = jnp.max(logits, axis=1, keepdims=True)
    s = jnp.sum(jnp.exp(logits - m), axis=1, keepdims=True)
    mc_ref[0] = m
    sc_ref[0] = s


def _b2_kernel(o_ref, w_ref, b_ref, mc_ref, sc_ref, pg_ref, tid_ref,
               pv_ref, pf_ref, m_sc, r_sc, *, v_total, chunk, nck, nc):
    cg = jnp.minimum(pl.program_id(0) * nck + pl.program_id(1), nc - 1)

    @pl.when(pl.program_id(1) == 0)
    def _():
        mall = mc_ref[...]
        mm = jnp.max(mall, axis=0)
        ss = jnp.sum(sc_ref[...] * jnp.exp(mall - mm[None]), axis=0)
        m_sc[...] = mm
        r_sc[...] = 1.0 / ss

    m = m_sc[...]
    rinv = r_sc[...]
    logits = lax.dot_general(o_ref[...], w_ref[...], (((1,), (1,)), ((), ())),
                             preferred_element_type=jnp.float32) + b_ref[...]
    jcol = lax.broadcasted_iota(jnp.int32, logits.shape, 1)
    logits = jnp.where(jcol < (v_total - cg * chunk), logits, -1e30)
    pv = jnp.exp(logits - m) * rinv
    pv_ref[...] = pv
    pg = pg_ref[:, 0:1]
    pf_ref[...] = pv * pg + (1.0 - pg) * tid_ref[...]


def kernel(input_token, last_decoder_hidden, encoder_states, full_input_var,
           emb_table, gru_w_ih, gru_w_hh, gru_b_ih, gru_b_hh,
           w_h, w_s, att_bias, attn_v, gen_w, gen_b,
           outh_w, outh_b, outv_w, outv_b):
    B, S, H = encoder_states.shape
    V, E = emb_table.shape
    PAD = 250
    VE = V + PAD
    NR = (VE + 127) // 128
    VEP = NR * 128
    CHUNK = 2048
    NC = 25
    NCK = 13
    BN = 16
    NBLK = (B // BN) // 2
    EB = B // 2

    f32 = jnp.float32
    cgc = lambda p, c: jnp.minimum(p * NCK + c, NC - 1)
    tok = input_token.reshape(B).astype(jnp.int32)
    emb3 = emb_table.reshape(V, 1, E)

    h_new, x = pl.pallas_call(
        _e_kernel,
        grid=(2, 1),
        in_specs=[
            pl.BlockSpec(memory_space=pltpu.SMEM),
            pl.BlockSpec(memory_space=pl.ANY),
            pl.BlockSpec((EB, H), lambda p, q: (p, 0)),
            pl.BlockSpec((3 * H, E), lambda p, q: (0, 0)),
            pl.BlockSpec((3 * H, H), lambda p, q: (0, 0)),
            pl.BlockSpec((1, 3 * H), lambda p, q: (0, 0)),
            pl.BlockSpec((1, 3 * H), lambda p, q: (0, 0)),
        ],
        out_specs=[
            pl.BlockSpec((EB, H), lambda p, q: (p, 0)),
            pl.BlockSpec((EB, E), lambda p, q: (p, 0)),
        ],
        out_shape=[
            jax.ShapeDtypeStruct((B, H), f32),
            jax.ShapeDtypeStruct((B, E), f32),
        ],
        scratch_shapes=[
            pltpu.VMEM((EB, 1, E), f32),
            pltpu.SemaphoreType.DMA,
        ],
        compiler_params=pltpu.CompilerParams(dimension_semantics=_SEM),
        name="embed_gru",
    )(tok, emb3, last_decoder_hidden, gru_w_ih, gru_w_hh,
      gru_b_ih.reshape(1, 3 * H), gru_b_hh.reshape(1, 3 * H))

    o, pgen_b, att_dist = pl.pallas_call(
        _a_kernel,
        grid=(2, NBLK),
        in_specs=[
            pl.BlockSpec((BN, S, H), lambda p, i: (p * NBLK + i, 0, 0)),
            pl.BlockSpec((BN, H), lambda p, i: (p * NBLK + i, 0)),
            pl.BlockSpec((BN, E), lambda p, i: (p * NBLK + i, 0)),
            pl.BlockSpec((1, H), lambda p, i: (0, 0)),
            pl.BlockSpec((1, H), lambda p, i: (0, 0)),
            pl.BlockSpec((H, 1), lambda p, i: (0, 0)),
            pl.BlockSpec(memory_space=pltpu.SMEM),
            pl.BlockSpec((1, H), lambda p, i: (0, 0)),
            pl.BlockSpec((1, H), lambda p, i: (0, 0)),
            pl.BlockSpec((1, E), lambda p, i: (0, 0)),
            pl.BlockSpec(memory_space=pltpu.SMEM),
            pl.BlockSpec((E, H), lambda p, i: (0, 0)),
            pl.BlockSpec((E, H), lambda p, i: (0, 0)),
            pl.BlockSpec((1, E), lambda p, i: (0, 0)),
        ],
        out_specs=[
            pl.BlockSpec((BN, E), lambda p, i: (p * NBLK + i, 0)),
            pl.BlockSpec((BN, 128), lambda p, i: (p * NBLK + i, 0)),
            pl.BlockSpec((BN, S), lambda p, i: (p * NBLK + i, 0)),
        ],
        out_shape=[
            jax.ShapeDtypeStruct((B, E), f32),
            jax.ShapeDtypeStruct((B, 128), f32),
            jax.ShapeDtypeStruct((B, S), f32),
        ],
        scratch_shapes=[
            pltpu.VMEM((S, BN), f32),
            pltpu.VMEM((BN, H), f32),
        ],
        compiler_params=pltpu.CompilerParams(dimension_semantics=_SEM),
        name="attn_pgen",
    )(encoder_states, h_new, x,
      w_h.reshape(1, H), w_s.reshape(1, H), attn_v.reshape(H, 1),
      att_bias.reshape(1, 1),
      gen_w[:, :H], gen_w[:, H:2 * H], gen_w[:, 2 * H:],
      gen_b.reshape(1, 1),
      outh_w[:, :H], outh_w[:, H:], outh_b.reshape(1, E))

    NRP = 400
    fiv = full_input_var.astype(jnp.int32)
    tid = pl.pallas_call(
        functools.partial(_c_kernel, s_len=S, bn=BN, nrp=NRP),
        grid=(2, NBLK),
        in_specs=[
            pl.BlockSpec((BN, S), lambda p, i: (p * NBLK + i, 0)),
            pl.BlockSpec((BN, S), lambda p, i: (p * NBLK + i, 0)),
            pl.BlockSpec((BN, S), lambda p, i: (p * NBLK + i, 0)),
        ],
        out_specs=pl.BlockSpec((BN * NRP, 128), lambda p, i: (p * NBLK + i, 0)),
        out_shape=jax.ShapeDtypeStruct((B * NRP, 128), f32),
        compiler_params=pltpu.CompilerParams(dimension_semantics=_SEM),
        name="ptr_scatter",
    )(fiv >> 7, fiv & 127, att_dist)
    tid2 = tid.reshape(B, NRP * 128)

    mC, sC = pl.pallas_call(
        functools.partial(_b1_kernel, v_total=V, chunk=CHUNK, nck=NCK, nc=NC),
        grid=(2, NCK),
        in_specs=[
            pl.BlockSpec((B, E), lambda p, c: (0, 0)),
            pl.BlockSpec((CHUNK, E), lambda p, c: (cgc(p, c), 0)),
            pl.BlockSpec((1, CHUNK), lambda p, c: (0, cgc(p, c))),
        ],
        out_specs=[
            pl.BlockSpec((1, B, 1), lambda p, c: (cgc(p, c), 0, 0)),
            pl.BlockSpec((1, B, 1), lambda p, c: (cgc(p, c), 0, 0)),
        ],
        out_shape=[
            jax.ShapeDtypeStruct((NC, B, 1), f32),
            jax.ShapeDtypeStruct((NC, B, 1), f32),
        ],
        compiler_params=pltpu.CompilerParams(dimension_semantics=_SEM),
        name="logit_stats",
    )(o, outv_w, outv_b.reshape(1, V))

    p_vocab, p_final = pl.pallas_call(
        functools.partial(_b2_kernel, v_total=V, chunk=CHUNK, nck=NCK, nc=NC),
        grid=(2, NCK),
        in_specs=[
            pl.BlockSpec((B, E), lambda p, c: (0, 0)),
            pl.BlockSpec((CHUNK, E), lambda p, c: (cgc(p, c), 0)),
            pl.BlockSpec((1, CHUNK), lambda p, c: (0, cgc(p, c))),
            pl.BlockSpec((NC, B, 1), lambda p, c: (0, 0, 0)),
            pl.BlockSpec((NC, B, 1), lambda p, c: (0, 0, 0)),
            pl.BlockSpec((B, 128), lambda p, c: (0, 0)),
            pl.BlockSpec((B, CHUNK), lambda p, c: (0, cgc(p, c))),
        ],
        out_specs=[
            pl.BlockSpec((B, CHUNK), lambda p, c: (0, cgc(p, c))),
            pl.BlockSpec((B, CHUNK), lambda p, c: (0, cgc(p, c))),
        ],
        out_shape=[
            jax.ShapeDtypeStruct((B, V), f32),
            jax.ShapeDtypeStruct((B, VE), f32),
        ],
        scratch_shapes=[
            pltpu.VMEM((B, 1), f32),
            pltpu.VMEM((B, 1), f32),
        ],
        compiler_params=pltpu.CompilerParams(dimension_semantics=_SEM),
        name="vocab_mix",
    )(o, outv_w, outv_b.reshape(1, V), mC, sC, pgen_b, tid2)

    p_gen = pgen_b[:, 0:1]
    return (h_new, p_final, p_gen, p_vocab, att_dist)


# speedup vs baseline: 1.6704x; 1.0565x over previous
import functools

import jax
import jax.numpy as jnp
from jax import lax
from jax.experimental import pallas as pl
from jax.experimental.pallas import tpu as pltpu

_SEM = ("parallel", "arbitrary")


def _e_kernel(tok_ref, emb_ref, h_ref, wih_ref, whh_ref, bih_ref, bhh_ref,
              hnew_ref, x_ref, x3, sem):
    nb = x3.shape[0]
    base = pl.program_id(0) * nb
    for i in range(nb):
        pltpu.make_async_copy(emb_ref.at[tok_ref[base + i]], x3.at[i], sem).start()
    for i in range(nb):
        pltpu.make_async_copy(emb_ref.at[tok_ref[base + i]], x3.at[i], sem).wait()
    x = x3[...].reshape(nb, x3.shape[2])
    x_ref[...] = x
    h = h_ref[...]
    gi = lax.dot_general(x, wih_ref[...], (((1,), (1,)), ((), ())),
                         preferred_element_type=jnp.float32) + bih_ref[...]
    gh = lax.dot_general(h, whh_ref[...], (((1,), (1,)), ((), ())),
                         preferred_element_type=jnp.float32) + bhh_ref[...]
    hh = h.shape[1]
    r = jax.nn.sigmoid(gi[:, :hh] + gh[:, :hh])
    z = jax.nn.sigmoid(gi[:, hh:2 * hh] + gh[:, hh:2 * hh])
    n = jnp.tanh(gi[:, 2 * hh:] + r * gh[:, 2 * hh:])
    hnew_ref[...] = (1.0 - z) * n + z * h


def _a_kernel(enc_ref, h_ref, x_ref, wh_ref, ws_ref, vT_ref, attb_ref,
              gwh_ref, gwc_ref, gwx_ref, genb_ref, ohwh_ref, ohwc_ref, ohb_ref,
              o_ref, pgen_ref, ad_ref, scT_s, ctx_s):
    bn = h_ref.shape[0]
    h = h_ref[...]
    hsum = ws_ref[...] * h + attb_ref[0, 0]
    wh = wh_ref[...]
    vT = vT_ref[...]
    for i in range(bn):
        encb = enc_ref[i]
        a = jnp.tanh(encb * wh + hsum[i:i + 1, :])
        scT_s[:, i:i + 1] = lax.dot_general(
            a, vT, (((1,), (0,)), ((), ())),
            preferred_element_type=jnp.float32)
    scores = jnp.transpose(scT_s[...])
    m = jnp.max(scores, axis=1, keepdims=True)
    e = jnp.exp(scores - m)
    ad = e * (1.0 / jnp.sum(e, axis=1, keepdims=True))
    ad_ref[...] = ad
    adT = jnp.transpose(ad)
    for i in range(bn):
        encb = enc_ref[i]
        ctx_s[i:i + 1, :] = jnp.sum(encb * adT[:, i:i + 1], axis=0, keepdims=True)
    ctx = ctx_s[...]
    o = (lax.dot_general(h, ohwh_ref[...], (((1,), (1,)), ((), ())),
                         preferred_element_type=jnp.float32)
         + lax.dot_general(ctx, ohwc_ref[...], (((1,), (1,)), ((), ())),
                           preferred_element_type=jnp.float32)
         + ohb_ref[...])
    o_ref[...] = o
    g = (lax.dot_general(h, gwh_ref[...], (((1,), (1,)), ((), ())),
                         preferred_element_type=jnp.float32)
         + lax.dot_general(ctx, gwc_ref[...], (((1,), (1,)), ((), ())),
                           preferred_element_type=jnp.float32)
         + lax.dot_general(x_ref[...], gwx_ref[...], (((1,), (1,)), ((), ())),
                           preferred_element_type=jnp.float32)
         + genb_ref[0, 0])
    pgen_ref[...] = jnp.broadcast_to(jax.nn.sigmoid(g), pgen_ref.shape)


def _c_kernel(q_ref, l_ref, ad_ref, tid_ref, *, s_len, bn, nrp):
    qT = jnp.transpose(q_ref[...])
    lT = jnp.transpose(l_ref[...])
    aT = jnp.transpose(ad_ref[...])
    iq = lax.broadcasted_iota(jnp.int32, (1, nrp), 1)
    il = lax.broadcasted_iota(jnp.int32, (1, 128), 1)
    for j in range(bn):
        pmat = jnp.where(qT[:, j:j + 1] == iq, aT[:, j:j + 1],
                         0.0).astype(jnp.bfloat16)
        momat = jnp.where(lT[:, j:j + 1] == il, 1.0, 0.0).astype(jnp.bfloat16)
        tb = lax.dot_general(pmat, momat, (((0,), (0,)), ((), ())),
                             preferred_element_type=jnp.float32)
        tid_ref[j * nrp:(j + 1) * nrp, :] = tb


def _b1_kernel(o_ref, w_ref, b_ref, mc_ref, sc_ref, *, v_total, chunk, nck, nc):
    cg = jnp.minimum(pl.program_id(0) * nck + pl.program_id(1), nc - 1)
    logits = lax.dot_general(o_ref[...], w_ref[...], (((1,), (1,)), ((), ())),
                             preferred_element_type=jnp.float32) + b_ref[...]
    jcol = lax.broadcasted_iota(jnp.int32, logits.shape, 1)
    logits = jnp.where(jcol < (v_total - cg * chunk), logits, -1e30)
    m = jnp.max(logits, axis=1, keepdims=True)
    s = jnp.sum(jnp.exp(logits - m), axis=1, keepdims=True)
    mc_ref[0] = m
    sc_ref[0] = s


def _b2_kernel(o_ref, w_ref, b_ref, mc_ref, sc_ref, pg_ref, tid_ref,
               pv_ref, pf_ref, m_sc, r_sc, *, v_total, chunk, nck, nc):
    cg = jnp.minimum(pl.program_id(0) * nck + pl.program_id(1), nc - 1)

    @pl.when(pl.program_id(1) == 0)
    def _():
        mall = mc_ref[...]
        mm = jnp.max(mall, axis=0)
        ss = jnp.sum(sc_ref[...] * jnp.exp(mall - mm[None]), axis=0)
        m_sc[...] = mm
        r_sc[...] = 1.0 / ss

    m = m_sc[...]
    rinv = r_sc[...]
    logits = lax.dot_general(o_ref[...], w_ref[...], (((1,), (1,)), ((), ())),
                             preferred_element_type=jnp.float32) + b_ref[...]
    jcol = lax.broadcasted_iota(jnp.int32, logits.shape, 1)
    logits = jnp.where(jcol < (v_total - cg * chunk), logits, -1e30)
    pv = jnp.exp(logits - m) * rinv
    pv_ref[...] = pv
    pg = pg_ref[:, 0:1]
    pf_ref[...] = pv * pg + (1.0 - pg) * tid_ref[...]


def kernel(input_token, last_decoder_hidden, encoder_states, full_input_var,
           emb_table, gru_w_ih, gru_w_hh, gru_b_ih, gru_b_hh,
           w_h, w_s, att_bias, attn_v, gen_w, gen_b,
           outh_w, outh_b, outv_w, outv_b):
    B, S, H = encoder_states.shape
    V, E = emb_table.shape
    PAD = 250
    VE = V + PAD
    NR = (VE + 127) // 128
    VEP = NR * 128
    CHUNK = 4096
    NC = 13
    NCK = 7
    BN = 16
    NBLK = (B // BN) // 2
    EB = B // 2

    f32 = jnp.float32
    cgc = lambda p, c: jnp.minimum(p * NCK + c, NC - 1)
    tok = input_token.reshape(B).astype(jnp.int32)
    emb3 = emb_table.reshape(V, 1, E)

    h_new, x = pl.pallas_call(
        _e_kernel,
        grid=(2, 1),
        in_specs=[
            pl.BlockSpec(memory_space=pltpu.SMEM),
            pl.BlockSpec(memory_space=pl.ANY),
            pl.BlockSpec((EB, H), lambda p, q: (p, 0)),
            pl.BlockSpec((3 * H, E), lambda p, q: (0, 0)),
            pl.BlockSpec((3 * H, H), lambda p, q: (0, 0)),
            pl.BlockSpec((1, 3 * H), lambda p, q: (0, 0)),
            pl.BlockSpec((1, 3 * H), lambda p, q: (0, 0)),
        ],
        out_specs=[
            pl.BlockSpec((EB, H), lambda p, q: (p, 0)),
            pl.BlockSpec((EB, E), lambda p, q: (p, 0)),
        ],
        out_shape=[
            jax.ShapeDtypeStruct((B, H), f32),
            jax.ShapeDtypeStruct((B, E), f32),
        ],
        scratch_shapes=[
            pltpu.VMEM((EB, 1, E), f32),
            pltpu.SemaphoreType.DMA,
        ],
        compiler_params=pltpu.CompilerParams(dimension_semantics=_SEM),
        name="embed_gru",
    )(tok, emb3, last_decoder_hidden, gru_w_ih, gru_w_hh,
      gru_b_ih.reshape(1, 3 * H), gru_b_hh.reshape(1, 3 * H))

    o, pgen_b, att_dist = pl.pallas_call(
        _a_kernel,
        grid=(2, NBLK),
        in_specs=[
            pl.BlockSpec((BN, S, H), lambda p, i: (p * NBLK + i, 0, 0)),
            pl.BlockSpec((BN, H), lambda p, i: (p * NBLK + i, 0)),
            pl.BlockSpec((BN, E), lambda p, i: (p * NBLK + i, 0)),
            pl.BlockSpec((1, H), lambda p, i: (0, 0)),
            pl.BlockSpec((1, H), lambda p, i: (0, 0)),
            pl.BlockSpec((H, 1), lambda p, i: (0, 0)),
            pl.BlockSpec(memory_space=pltpu.SMEM),
            pl.BlockSpec((1, H), lambda p, i: (0, 0)),
            pl.BlockSpec((1, H), lambda p, i: (0, 0)),
            pl.BlockSpec((1, E), lambda p, i: (0, 0)),
            pl.BlockSpec(memory_space=pltpu.SMEM),
            pl.BlockSpec((E, H), lambda p, i: (0, 0)),
            pl.BlockSpec((E, H), lambda p, i: (0, 0)),
            pl.BlockSpec((1, E), lambda p, i: (0, 0)),
        ],
        out_specs=[
            pl.BlockSpec((BN, E), lambda p, i: (p * NBLK + i, 0)),
            pl.BlockSpec((BN, 128), lambda p, i: (p * NBLK + i, 0)),
            pl.BlockSpec((BN, S), lambda p, i: (p * NBLK + i, 0)),
        ],
        out_shape=[
            jax.ShapeDtypeStruct((B, E), f32),
            jax.ShapeDtypeStruct((B, 128), f32),
            jax.ShapeDtypeStruct((B, S), f32),
        ],
        scratch_shapes=[
            pltpu.VMEM((S, BN), f32),
            pltpu.VMEM((BN, H), f32),
        ],
        compiler_params=pltpu.CompilerParams(dimension_semantics=_SEM),
        name="attn_pgen",
    )(encoder_states, h_new, x,
      w_h.reshape(1, H), w_s.reshape(1, H), attn_v.reshape(H, 1),
      att_bias.reshape(1, 1),
      gen_w[:, :H], gen_w[:, H:2 * H], gen_w[:, 2 * H:],
      gen_b.reshape(1, 1),
      outh_w[:, :H], outh_w[:, H:], outh_b.reshape(1, E))

    NRP = 400
    fiv = full_input_var.astype(jnp.int32)
    tid = pl.pallas_call(
        functools.partial(_c_kernel, s_len=S, bn=BN, nrp=NRP),
        grid=(2, NBLK),
        in_specs=[
            pl.BlockSpec((BN, S), lambda p, i: (p * NBLK + i, 0)),
            pl.BlockSpec((BN, S), lambda p, i: (p * NBLK + i, 0)),
            pl.BlockSpec((BN, S), lambda p, i: (p * NBLK + i, 0)),
        ],
        out_specs=pl.BlockSpec((BN * NRP, 128), lambda p, i: (p * NBLK + i, 0)),
        out_shape=jax.ShapeDtypeStruct((B * NRP, 128), f32),
        compiler_params=pltpu.CompilerParams(dimension_semantics=_SEM),
        name="ptr_scatter",
    )(fiv >> 7, fiv & 127, att_dist)
    tid2 = tid.reshape(B, NRP * 128)

    mC, sC = pl.pallas_call(
        functools.partial(_b1_kernel, v_total=V, chunk=CHUNK, nck=NCK, nc=NC),
        grid=(2, NCK),
        in_specs=[
            pl.BlockSpec((B, E), lambda p, c: (0, 0)),
            pl.BlockSpec((CHUNK, E), lambda p, c: (cgc(p, c), 0)),
            pl.BlockSpec((1, CHUNK), lambda p, c: (0, cgc(p, c))),
        ],
        out_specs=[
            pl.BlockSpec((1, B, 1), lambda p, c: (cgc(p, c), 0, 0)),
            pl.BlockSpec((1, B, 1), lambda p, c: (cgc(p, c), 0, 0)),
        ],
        out_shape=[
            jax.ShapeDtypeStruct((NC, B, 1), f32),
            jax.ShapeDtypeStruct((NC, B, 1), f32),
        ],
        compiler_params=pltpu.CompilerParams(dimension_semantics=_SEM),
        name="logit_stats",
    )(o, outv_w, outv_b.reshape(1, V))

    p_vocab, p_final = pl.pallas_call(
        functools.partial(_b2_kernel, v_total=V, chunk=CHUNK, nck=NCK, nc=NC),
        grid=(2, NCK),
        in_specs=[
            pl.BlockSpec((B, E), lambda p, c: (0, 0)),
            pl.BlockSpec((CHUNK, E), lambda p, c: (cgc(p, c), 0)),
            pl.BlockSpec((1, CHUNK), lambda p, c: (0, cgc(p, c))),
            pl.BlockSpec((NC, B, 1), lambda p, c: (0, 0, 0)),
            pl.BlockSpec((NC, B, 1), lambda p, c: (0, 0, 0)),
            pl.BlockSpec((B, 128), lambda p, c: (0, 0)),
            pl.BlockSpec((B, CHUNK), lambda p, c: (0, cgc(p, c))),
        ],
        out_specs=[
            pl.BlockSpec((B, CHUNK), lambda p, c: (0, cgc(p, c))),
            pl.BlockSpec((B, CHUNK), lambda p, c: (0, cgc(p, c))),
        ],
        out_shape=[
            jax.ShapeDtypeStruct((B, V), f32),
            jax.ShapeDtypeStruct((B, VE), f32),
        ],
        scratch_shapes=[
            pltpu.VMEM((B, 1), f32),
            pltpu.VMEM((B, 1), f32),
        ],
        compiler_params=pltpu.CompilerParams(dimension_semantics=_SEM),
        name="vocab_mix",
    )(o, outv_w, outv_b.reshape(1, V), mC, sC, pgen_b, tid2)

    p_gen = pgen_b[:, 0:1]
    return (h_new, p_final, p_gen, p_vocab, att_dist)


# speedup vs baseline: 1.7681x; 1.0585x over previous
import functools

import jax
import jax.numpy as jnp
from jax import lax
from jax.experimental import pallas as pl
from jax.experimental.pallas import tpu as pltpu

_SEM = ("parallel", "arbitrary")


def _e_kernel(tok_ref, emb_ref, h_ref, wih_ref, whh_ref, bih_ref, bhh_ref,
              hnew_ref, x_ref, x3, sem):
    nb = x3.shape[0]
    base = pl.program_id(0) * nb
    for i in range(nb):
        pltpu.make_async_copy(emb_ref.at[tok_ref[base + i]], x3.at[i], sem).start()
    for i in range(nb):
        pltpu.make_async_copy(emb_ref.at[tok_ref[base + i]], x3.at[i], sem).wait()
    x = x3[...].reshape(nb, x3.shape[2])
    x_ref[...] = x
    h = h_ref[...]
    gi = lax.dot_general(x, wih_ref[...], (((1,), (1,)), ((), ())),
                         preferred_element_type=jnp.float32) + bih_ref[...]
    gh = lax.dot_general(h, whh_ref[...], (((1,), (1,)), ((), ())),
                         preferred_element_type=jnp.float32) + bhh_ref[...]
    hh = h.shape[1]
    r = jax.nn.sigmoid(gi[:, :hh] + gh[:, :hh])
    z = jax.nn.sigmoid(gi[:, hh:2 * hh] + gh[:, hh:2 * hh])
    n = jnp.tanh(gi[:, 2 * hh:] + r * gh[:, 2 * hh:])
    hnew_ref[...] = (1.0 - z) * n + z * h


def _a_kernel(enc_ref, h_ref, x_ref, q_ref, l_ref, wh_ref, ws_ref, vT_ref,
              attb_ref, gwh_ref, gwc_ref, gwx_ref, genb_ref, ohwh_ref,
              ohwc_ref, ohb_ref,
              o_ref, pgen_ref, ad_ref, tid_ref, scT_s, ctx_s, *, nrp):
    bn = h_ref.shape[0]
    h = h_ref[...]
    hsum = ws_ref[...] * h + attb_ref[0, 0]
    wh = wh_ref[...]
    vT = vT_ref[...]
    for i in range(bn):
        encb = enc_ref[i]
        a = jnp.tanh(encb * wh + hsum[i:i + 1, :])
        scT_s[:, i:i + 1] = lax.dot_general(
            a, vT, (((1,), (0,)), ((), ())),
            preferred_element_type=jnp.float32)
    scores = jnp.transpose(scT_s[...])
    m = jnp.max(scores, axis=1, keepdims=True)
    e = jnp.exp(scores - m)
    ad = e * (1.0 / jnp.sum(e, axis=1, keepdims=True))
    ad_ref[...] = ad
    adT = jnp.transpose(ad)
    for i in range(bn):
        encb = enc_ref[i]
        ctx_s[i:i + 1, :] = jnp.sum(encb * adT[:, i:i + 1], axis=0, keepdims=True)
    ctx = ctx_s[...]
    o = (lax.dot_general(h, ohwh_ref[...], (((1,), (1,)), ((), ())),
                         preferred_element_type=jnp.float32)
         + lax.dot_general(ctx, ohwc_ref[...], (((1,), (1,)), ((), ())),
                           preferred_element_type=jnp.float32)
         + ohb_ref[...])
    o_ref[...] = o
    g = (lax.dot_general(h, gwh_ref[...], (((1,), (1,)), ((), ())),
                         preferred_element_type=jnp.float32)
         + lax.dot_general(ctx, gwc_ref[...], (((1,), (1,)), ((), ())),
                           preferred_element_type=jnp.float32)
         + lax.dot_general(x_ref[...], gwx_ref[...], (((1,), (1,)), ((), ())),
                           preferred_element_type=jnp.float32)
         + genb_ref[0, 0])
    pgen_ref[...] = jnp.broadcast_to(jax.nn.sigmoid(g), pgen_ref.shape)
    qT = jnp.transpose(q_ref[...])
    lT = jnp.transpose(l_ref[...])
    iq = lax.broadcasted_iota(jnp.int32, (1, nrp), 1)
    il = lax.broadcasted_iota(jnp.int32, (1, 128), 1)
    for j in range(bn):
        pmat = jnp.where(qT[:, j:j + 1] == iq, adT[:, j:j + 1],
                         0.0).astype(jnp.bfloat16)
        momat = jnp.where(lT[:, j:j + 1] == il, 1.0, 0.0).astype(jnp.bfloat16)
        tb = lax.dot_general(pmat, momat, (((0,), (0,)), ((), ())),
                             preferred_element_type=jnp.float32)
        tid_ref[j * nrp:(j + 1) * nrp, :] = tb


def _b1_kernel(o_ref, w_ref, b_ref, mc_ref, sc_ref, *, v_total, chunk, nck, nc):
    cg = jnp.minimum(pl.program_id(0) * nck + pl.program_id(1), nc - 1)
    logits = lax.dot_general(o_ref[...], w_ref[...], (((1,), (1,)), ((), ())),
                             preferred_element_type=jnp.float32) + b_ref[...]
    jcol = lax.broadcasted_iota(jnp.int32, logits.shape, 1)
    logits = jnp.where(jcol < (v_total - cg * chunk), logits, -1e30)
    m = jnp.max(logits, axis=1, keepdims=True)
    s = jnp.sum(jnp.exp(logits - m), axis=1, keepdims=True)
    mc_ref[0] = m
    sc_ref[0] = s


def _b2_kernel(o_ref, w_ref, b_ref, mc_ref, sc_ref, pg_ref, tid_ref,
               pv_ref, pf_ref, m_sc, r_sc, *, v_total, chunk, nck, nc):
    cg = jnp.minimum(pl.program_id(0) * nck + pl.program_id(1), nc - 1)

    @pl.when(pl.program_id(1) == 0)
    def _():
        mall = mc_ref[...]
        mm = jnp.max(mall, axis=0)
        ss = jnp.sum(sc_ref[...] * jnp.exp(mall - mm[None]), axis=0)
        m_sc[...] = mm
        r_sc[...] = 1.0 / ss

    m = m_sc[...]
    rinv = r_sc[...]
    logits = lax.dot_general(o_ref[...], w_ref[...], (((1,), (1,)), ((), ())),
                             preferred_element_type=jnp.float32) + b_ref[...]
    jcol = lax.broadcasted_iota(jnp.int32, logits.shape, 1)
    logits = jnp.where(jcol < (v_total - cg * chunk), logits, -1e30)
    pv = jnp.exp(logits - m) * rinv
    pv_ref[...] = pv
    pg = pg_ref[:, 0:1]
    pf_ref[...] = pv * pg + (1.0 - pg) * tid_ref[...]


def kernel(input_token, last_decoder_hidden, encoder_states, full_input_var,
           emb_table, gru_w_ih, gru_w_hh, gru_b_ih, gru_b_hh,
           w_h, w_s, att_bias, attn_v, gen_w, gen_b,
           outh_w, outh_b, outv_w, outv_b):
    B, S, H = encoder_states.shape
    V, E = emb_table.shape
    PAD = 250
    VE = V + PAD
    NR = (VE + 127) // 128
    VEP = NR * 128
    CHUNK = 4096
    NC = 13
    NCK = 7
    BN = 16
    NBLK = (B // BN) // 2
    EB = B // 2

    f32 = jnp.float32
    cgc = lambda p, c: jnp.minimum(p * NCK + c, NC - 1)
    tok = input_token.reshape(B).astype(jnp.int32)
    emb3 = emb_table.reshape(V, 1, E)

    h_new, x = pl.pallas_call(
        _e_kernel,
        grid=(2, 1),
        in_specs=[
            pl.BlockSpec(memory_space=pltpu.SMEM),
            pl.BlockSpec(memory_space=pl.ANY),
            pl.BlockSpec((EB, H), lambda p, q: (p, 0)),
            pl.BlockSpec((3 * H, E), lambda p, q: (0, 0)),
            pl.BlockSpec((3 * H, H), lambda p, q: (0, 0)),
            pl.BlockSpec((1, 3 * H), lambda p, q: (0, 0)),
            pl.BlockSpec((1, 3 * H), lambda p, q: (0, 0)),
        ],
        out_specs=[
            pl.BlockSpec((EB, H), lambda p, q: (p, 0)),
            pl.BlockSpec((EB, E), lambda p, q: (p, 0)),
        ],
        out_shape=[
            jax.ShapeDtypeStruct((B, H), f32),
            jax.ShapeDtypeStruct((B, E), f32),
        ],
        scratch_shapes=[
            pltpu.VMEM((EB, 1, E), f32),
            pltpu.SemaphoreType.DMA,
        ],
        compiler_params=pltpu.CompilerParams(dimension_semantics=_SEM),
        name="embed_gru",
    )(tok, emb3, last_decoder_hidden, gru_w_ih, gru_w_hh,
      gru_b_ih.reshape(1, 3 * H), gru_b_hh.reshape(1, 3 * H))

    NRP = 400
    fiv = full_input_var.astype(jnp.int32)
    o, pgen_b, att_dist, tid = pl.pallas_call(
        functools.partial(_a_kernel, nrp=NRP),
        grid=(2, NBLK),
        in_specs=[
            pl.BlockSpec((BN, S, H), lambda p, i: (p * NBLK + i, 0, 0)),
            pl.BlockSpec((BN, H), lambda p, i: (p * NBLK + i, 0)),
            pl.BlockSpec((BN, E), lambda p, i: (p * NBLK + i, 0)),
            pl.BlockSpec((BN, S), lambda p, i: (p * NBLK + i, 0)),
            pl.BlockSpec((BN, S), lambda p, i: (p * NBLK + i, 0)),
            pl.BlockSpec((1, H), lambda p, i: (0, 0)),
            pl.BlockSpec((1, H), lambda p, i: (0, 0)),
            pl.BlockSpec((H, 1), lambda p, i: (0, 0)),
            pl.BlockSpec(memory_space=pltpu.SMEM),
            pl.BlockSpec((1, H), lambda p, i: (0, 0)),
            pl.BlockSpec((1, H), lambda p, i: (0, 0)),
            pl.BlockSpec((1, E), lambda p, i: (0, 0)),
            pl.BlockSpec(memory_space=pltpu.SMEM),
            pl.BlockSpec((E, H), lambda p, i: (0, 0)),
            pl.BlockSpec((E, H), lambda p, i: (0, 0)),
            pl.BlockSpec((1, E), lambda p, i: (0, 0)),
        ],
        out_specs=[
            pl.BlockSpec((BN, E), lambda p, i: (p * NBLK + i, 0)),
            pl.BlockSpec((BN, 128), lambda p, i: (p * NBLK + i, 0)),
            pl.BlockSpec((BN, S), lambda p, i: (p * NBLK + i, 0)),
            pl.BlockSpec((BN * NRP, 128), lambda p, i: (p * NBLK + i, 0)),
        ],
        out_shape=[
            jax.ShapeDtypeStruct((B, E), f32),
            jax.ShapeDtypeStruct((B, 128), f32),
            jax.ShapeDtypeStruct((B, S), f32),
            jax.ShapeDtypeStruct((B * NRP, 128), f32),
        ],
        scratch_shapes=[
            pltpu.VMEM((S, BN), f32),
            pltpu.VMEM((BN, H), f32),
        ],
        compiler_params=pltpu.CompilerParams(dimension_semantics=_SEM),
        name="attn_pgen",
    )(encoder_states, h_new, x, fiv >> 7, fiv & 127,
      w_h.reshape(1, H), w_s.reshape(1, H), attn_v.reshape(H, 1),
      att_bias.reshape(1, 1),
      gen_w[:, :H], gen_w[:, H:2 * H], gen_w[:, 2 * H:],
      gen_b.reshape(1, 1),
      outh_w[:, :H], outh_w[:, H:], outh_b.reshape(1, E))

    tid2 = tid.reshape(B, NRP * 128)

    mC, sC = pl.pallas_call(
        functools.partial(_b1_kernel, v_total=V, chunk=CHUNK, nck=NCK, nc=NC),
        grid=(2, NCK),
        in_specs=[
            pl.BlockSpec((B, E), lambda p, c: (0, 0)),
            pl.BlockSpec((CHUNK, E), lambda p, c: (cgc(p, c), 0)),
            pl.BlockSpec((1, CHUNK), lambda p, c: (0, cgc(p, c))),
        ],
        out_specs=[
            pl.BlockSpec((1, B, 1), lambda p, c: (cgc(p, c), 0, 0)),
            pl.BlockSpec((1, B, 1), lambda p, c: (cgc(p, c), 0, 0)),
        ],
        out_shape=[
            jax.ShapeDtypeStruct((NC, B, 1), f32),
            jax.ShapeDtypeStruct((NC, B, 1), f32),
        ],
        compiler_params=pltpu.CompilerParams(dimension_semantics=_SEM),
        name="logit_stats",
    )(o, outv_w, outv_b.reshape(1, V))

    p_vocab, p_final = pl.pallas_call(
        functools.partial(_b2_kernel, v_total=V, chunk=CHUNK, nck=NCK, nc=NC),
        grid=(2, NCK),
        in_specs=[
            pl.BlockSpec((B, E), lambda p, c: (0, 0)),
            pl.BlockSpec((CHUNK, E), lambda p, c: (cgc(p, c), 0)),
            pl.BlockSpec((1, CHUNK), lambda p, c: (0, cgc(p, c))),
            pl.BlockSpec((NC, B, 1), lambda p, c: (0, 0, 0)),
            pl.BlockSpec((NC, B, 1), lambda p, c: (0, 0, 0)),
            pl.BlockSpec((B, 128), lambda p, c: (0, 0)),
            pl.BlockSpec((B, CHUNK), lambda p, c: (0, cgc(p, c))),
        ],
        out_specs=[
            pl.BlockSpec((B, CHUNK), lambda p, c: (0, cgc(p, c))),
            pl.BlockSpec((B, CHUNK), lambda p, c: (0, cgc(p, c))),
        ],
        out_shape=[
            jax.ShapeDtypeStruct((B, V), f32),
            jax.ShapeDtypeStruct((B, VE), f32),
        ],
        scratch_shapes=[
            pltpu.VMEM((B, 1), f32),
            pltpu.VMEM((B, 1), f32),
        ],
        compiler_params=pltpu.CompilerParams(dimension_semantics=_SEM),
        name="vocab_mix",
    )(o, outv_w, outv_b.reshape(1, V), mC, sC, pgen_b, tid2)

    p_gen = pgen_b[:, 0:1]
    return (h_new, p_final, p_gen, p_vocab, att_dist)


# speedup vs baseline: 1.7849x; 1.0095x over previous
import functools

import jax
import jax.numpy as jnp
from jax import lax
from jax.experimental import pallas as pl
from jax.experimental.pallas import tpu as pltpu

_SEM = ("parallel", "arbitrary")


def _e_kernel(tok_ref, emb_ref, h_ref, wih_ref, whh_ref, bih_ref, bhh_ref,
              hnew_ref, x_ref, x3, sem):
    nb = x3.shape[0]
    base = pl.program_id(0) * nb
    for i in range(nb):
        pltpu.make_async_copy(emb_ref.at[tok_ref[base + i]], x3.at[i], sem).start()
    for i in range(nb):
        pltpu.make_async_copy(emb_ref.at[tok_ref[base + i]], x3.at[i], sem).wait()
    x = x3[...].reshape(nb, x3.shape[2])
    x_ref[...] = x
    h = h_ref[...]
    gi = lax.dot_general(x, wih_ref[...], (((1,), (1,)), ((), ())),
                         preferred_element_type=jnp.float32) + bih_ref[...]
    gh = lax.dot_general(h, whh_ref[...], (((1,), (1,)), ((), ())),
                         preferred_element_type=jnp.float32) + bhh_ref[...]
    hh = h.shape[1]
    r = jax.nn.sigmoid(gi[:, :hh] + gh[:, :hh])
    z = jax.nn.sigmoid(gi[:, hh:2 * hh] + gh[:, hh:2 * hh])
    n = jnp.tanh(gi[:, 2 * hh:] + r * gh[:, 2 * hh:])
    hnew_ref[...] = (1.0 - z) * n + z * h


def _a_kernel(enc_ref, h_ref, x_ref, q_ref, l_ref, wh_ref, ws_ref, vT_ref,
              attb_ref, gwh_ref, gwc_ref, gwx_ref, genb_ref, ohwh_ref,
              ohwc_ref, ohb_ref,
              o_ref, pgen_ref, ad_ref, tid_ref, scT_s, ctx_s, *, nrp):
    bn = h_ref.shape[0]
    h = h_ref[...]
    hsum = ws_ref[...] * h + attb_ref[0, 0]
    wh = wh_ref[...]
    vT = vT_ref[...]
    for i in range(bn):
        encb = enc_ref[i]
        a = jnp.tanh(encb * wh + hsum[i:i + 1, :])
        scT_s[:, i:i + 1] = lax.dot_general(
            a, vT, (((1,), (0,)), ((), ())),
            preferred_element_type=jnp.float32)
    scores = jnp.transpose(scT_s[...])
    m = jnp.max(scores, axis=1, keepdims=True)
    e = jnp.exp(scores - m)
    ad = e * (1.0 / jnp.sum(e, axis=1, keepdims=True))
    ad_ref[...] = ad
    adT = jnp.transpose(ad)
    for i in range(bn):
        encb = enc_ref[i]
        ctx_s[i:i + 1, :] = jnp.sum(encb * adT[:, i:i + 1], axis=0, keepdims=True)
    ctx = ctx_s[...]
    o = (lax.dot_general(h, ohwh_ref[...], (((1,), (1,)), ((), ())),
                         preferred_element_type=jnp.float32)
         + lax.dot_general(ctx, ohwc_ref[...], (((1,), (1,)), ((), ())),
                           preferred_element_type=jnp.float32)
         + ohb_ref[...])
    o_ref[...] = o
    g = (lax.dot_general(h, gwh_ref[...], (((1,), (1,)), ((), ())),
                         preferred_element_type=jnp.float32)
         + lax.dot_general(ctx, gwc_ref[...], (((1,), (1,)), ((), ())),
                           preferred_element_type=jnp.float32)
         + lax.dot_general(x_ref[...], gwx_ref[...], (((1,), (1,)), ((), ())),
                           preferred_element_type=jnp.float32)
         + genb_ref[0, 0])
    pgen_ref[...] = jnp.broadcast_to(jax.nn.sigmoid(g), pgen_ref.shape)
    qT = jnp.transpose(q_ref[...])
    lT = jnp.transpose(l_ref[...])
    iq = lax.broadcasted_iota(jnp.int32, (1, nrp), 1)
    il = lax.broadcasted_iota(jnp.int32, (1, 128), 1)
    for j in range(bn):
        pmat = jnp.where(qT[:, j:j + 1] == iq, adT[:, j:j + 1],
                         0.0).astype(jnp.bfloat16)
        momat = jnp.where(lT[:, j:j + 1] == il, 1.0, 0.0).astype(jnp.bfloat16)
        tb = lax.dot_general(pmat, momat, (((0,), (0,)), ((), ())),
                             preferred_element_type=jnp.float32)
        tid_ref[j * nrp:(j + 1) * nrp, :] = tb


def _b_kernel(o_ref, w_ref, b_ref, pg_ref, tid_ref, pv_ref, pf_ref,
              m_sc, s_sc, *, v_total, chunk, nc):
    step = pl.program_id(0)
    cg = jnp.where(step < nc, step, step - nc)
    logits = lax.dot_general(o_ref[...], w_ref[...], (((1,), (1,)), ((), ())),
                             preferred_element_type=jnp.float32) + b_ref[...]
    jcol = lax.broadcasted_iota(jnp.int32, logits.shape, 1)
    logits = jnp.where(jcol < (v_total - cg * chunk), logits, -1e30)

    @pl.when(step == 0)
    def _():
        m_sc[...] = jnp.full_like(m_sc, -3e38)
        s_sc[...] = jnp.zeros_like(s_sc)

    @pl.when(step < nc)
    def _():
        mc = jnp.max(logits, axis=1, keepdims=True)
        mn = jnp.maximum(m_sc[...], mc)
        s_sc[...] = (s_sc[...] * jnp.exp(m_sc[...] - mn)
                     + jnp.sum(jnp.exp(logits - mn), axis=1, keepdims=True))
        m_sc[...] = mn

    @pl.when(step >= nc)
    def _():
        pv = jnp.exp(logits - m_sc[...]) * (1.0 / s_sc[...])
        pv_ref[...] = pv
        pg = pg_ref[:, 0:1]
        pf_ref[...] = pv * pg + (1.0 - pg) * tid_ref[...]


def kernel(input_token, last_decoder_hidden, encoder_states, full_input_var,
           emb_table, gru_w_ih, gru_w_hh, gru_b_ih, gru_b_hh,
           w_h, w_s, att_bias, attn_v, gen_w, gen_b,
           outh_w, outh_b, outv_w, outv_b):
    B, S, H = encoder_states.shape
    V, E = emb_table.shape
    PAD = 250
    VE = V + PAD
    NR = (VE + 127) // 128
    VEP = NR * 128
    CHUNK = 4096
    NC = 13
    BN = 16
    NBLK = (B // BN) // 2
    EB = B // 2

    f32 = jnp.float32
    cgc = lambda p, c: jnp.minimum(p * NCK + c, NC - 1)
    tok = input_token.reshape(B).astype(jnp.int32)
    emb3 = emb_table.reshape(V, 1, E)

    h_new, x = pl.pallas_call(
        _e_kernel,
        grid=(2, 1),
        in_specs=[
            pl.BlockSpec(memory_space=pltpu.SMEM),
            pl.BlockSpec(memory_space=pl.ANY),
            pl.BlockSpec((EB, H), lambda p, q: (p, 0)),
            pl.BlockSpec((3 * H, E), lambda p, q: (0, 0)),
            pl.BlockSpec((3 * H, H), lambda p, q: (0, 0)),
            pl.BlockSpec((1, 3 * H), lambda p, q: (0, 0)),
            pl.BlockSpec((1, 3 * H), lambda p, q: (0, 0)),
        ],
        out_specs=[
            pl.BlockSpec((EB, H), lambda p, q: (p, 0)),
            pl.BlockSpec((EB, E), lambda p, q: (p, 0)),
        ],
        out_shape=[
            jax.ShapeDtypeStruct((B, H), f32),
            jax.ShapeDtypeStruct((B, E), f32),
        ],
        scratch_shapes=[
            pltpu.VMEM((EB, 1, E), f32),
            pltpu.SemaphoreType.DMA,
        ],
        compiler_params=pltpu.CompilerParams(dimension_semantics=_SEM),
        name="embed_gru",
    )(tok, emb3, last_decoder_hidden, gru_w_ih, gru_w_hh,
      gru_b_ih.reshape(1, 3 * H), gru_b_hh.reshape(1, 3 * H))

    NRP = 400
    fiv = full_input_var.astype(jnp.int32)
    o, pgen_b, att_dist, tid = pl.pallas_call(
        functools.partial(_a_kernel, nrp=NRP),
        grid=(2, NBLK),
        in_specs=[
            pl.BlockSpec((BN, S, H), lambda p, i: (p * NBLK + i, 0, 0)),
            pl.BlockSpec((BN, H), lambda p, i: (p * NBLK + i, 0)),
            pl.BlockSpec((BN, E), lambda p, i: (p * NBLK + i, 0)),
            pl.BlockSpec((BN, S), lambda p, i: (p * NBLK + i, 0)),
            pl.BlockSpec((BN, S), lambda p, i: (p * NBLK + i, 0)),
            pl.BlockSpec((1, H), lambda p, i: (0, 0)),
            pl.BlockSpec((1, H), lambda p, i: (0, 0)),
            pl.BlockSpec((H, 1), lambda p, i: (0, 0)),
            pl.BlockSpec(memory_space=pltpu.SMEM),
            pl.BlockSpec((1, H), lambda p, i: (0, 0)),
            pl.BlockSpec((1, H), lambda p, i: (0, 0)),
            pl.BlockSpec((1, E), lambda p, i: (0, 0)),
            pl.BlockSpec(memory_space=pltpu.SMEM),
            pl.BlockSpec((E, H), lambda p, i: (0, 0)),
            pl.BlockSpec((E, H), lambda p, i: (0, 0)),
            pl.BlockSpec((1, E), lambda p, i: (0, 0)),
        ],
        out_specs=[
            pl.BlockSpec((BN, E), lambda p, i: (p * NBLK + i, 0)),
            pl.BlockSpec((BN, 128), lambda p, i: (p * NBLK + i, 0)),
            pl.BlockSpec((BN, S), lambda p, i: (p * NBLK + i, 0)),
            pl.BlockSpec((BN * NRP, 128), lambda p, i: (p * NBLK + i, 0)),
        ],
        out_shape=[
            jax.ShapeDtypeStruct((B, E), f32),
            jax.ShapeDtypeStruct((B, 128), f32),
            jax.ShapeDtypeStruct((B, S), f32),
            jax.ShapeDtypeStruct((B * NRP, 128), f32),
        ],
        scratch_shapes=[
            pltpu.VMEM((S, BN), f32),
            pltpu.VMEM((BN, H), f32),
        ],
        compiler_params=pltpu.CompilerParams(dimension_semantics=_SEM),
        name="attn_pgen",
    )(encoder_states, h_new, x, fiv >> 7, fiv & 127,
      w_h.reshape(1, H), w_s.reshape(1, H), attn_v.reshape(H, 1),
      att_bias.reshape(1, 1),
      gen_w[:, :H], gen_w[:, H:2 * H], gen_w[:, 2 * H:],
      gen_b.reshape(1, 1),
      outh_w[:, :H], outh_w[:, H:], outh_b.reshape(1, E))

    tid2 = tid.reshape(B, NRP * 128)

    def wb_idx(s):
        return jnp.where(s < NC, s, s - NC)

    def ph2_idx(s):
        return jnp.where(s < NC, 0, s - NC)

    p_vocab, p_final = pl.pallas_call(
        functools.partial(_b_kernel, v_total=V, chunk=CHUNK, nc=NC),
        grid=(2 * NC,),
        in_specs=[
            pl.BlockSpec((B, E), lambda s: (0, 0)),
            pl.BlockSpec((CHUNK, E), lambda s: (wb_idx(s), 0)),
            pl.BlockSpec((1, CHUNK), lambda s: (0, wb_idx(s))),
            pl.BlockSpec((B, 128), lambda s: (0, 0)),
            pl.BlockSpec((B, CHUNK), lambda s: (0, ph2_idx(s))),
        ],
        out_specs=[
            pl.BlockSpec((B, CHUNK), lambda s: (0, ph2_idx(s))),
            pl.BlockSpec((B, CHUNK), lambda s: (0, ph2_idx(s))),
        ],
        out_shape=[
            jax.ShapeDtypeStruct((B, V), f32),
            jax.ShapeDtypeStruct((B, VE), f32),
        ],
        scratch_shapes=[
            pltpu.VMEM((B, 1), f32),
            pltpu.VMEM((B, 1), f32),
        ],
        compiler_params=pltpu.CompilerParams(
            dimension_semantics=("arbitrary",)),
        name="vocab_softmax_mix",
    )(o, outv_w, outv_b.reshape(1, V), pgen_b, tid2)

    p_gen = pgen_b[:, 0:1]
    return (h_new, p_final, p_gen, p_vocab, att_dist)


# speedup vs baseline: 1.8028x; 1.0101x over previous
import functools

import jax
import jax.numpy as jnp
from jax import lax
from jax.experimental import pallas as pl
from jax.experimental.pallas import tpu as pltpu

_SEM = ("parallel", "arbitrary")


def _e_kernel(tok_ref, emb_ref, h_ref, wih_ref, whh_ref, bih_ref, bhh_ref,
              hnew_ref, x_ref, x3, sem):
    nb = x3.shape[0]
    base = pl.program_id(0) * nb
    for i in range(nb):
        pltpu.make_async_copy(emb_ref.at[tok_ref[base + i]], x3.at[i], sem).start()
    for i in range(nb):
        pltpu.make_async_copy(emb_ref.at[tok_ref[base + i]], x3.at[i], sem).wait()
    x = x3[...].reshape(nb, x3.shape[2])
    x_ref[...] = x
    h = h_ref[...]
    gi = lax.dot_general(x, wih_ref[...], (((1,), (1,)), ((), ())),
                         preferred_element_type=jnp.float32) + bih_ref[...]
    gh = lax.dot_general(h, whh_ref[...], (((1,), (1,)), ((), ())),
                         preferred_element_type=jnp.float32) + bhh_ref[...]
    hh = h.shape[1]
    r = jax.nn.sigmoid(gi[:, :hh] + gh[:, :hh])
    z = jax.nn.sigmoid(gi[:, hh:2 * hh] + gh[:, hh:2 * hh])
    n = jnp.tanh(gi[:, 2 * hh:] + r * gh[:, 2 * hh:])
    hnew_ref[...] = (1.0 - z) * n + z * h


def _a_kernel(enc_ref, h_ref, x_ref, q_ref, l_ref, wh_ref, ws_ref, vT_ref,
              attb_ref, gwh_ref, gwc_ref, gwx_ref, genb_ref, ohwh_ref,
              ohwc_ref, ohb_ref,
              o_ref, pgen_ref, ad_ref, tid_ref, scT_s, ctx_s, *, nrp):
    bn = h_ref.shape[0]
    h = h_ref[...]
    hsum = ws_ref[...] * h + attb_ref[0, 0]
    wh = wh_ref[...]
    vT = vT_ref[...]
    for i in range(bn):
        encb = enc_ref[i]
        a = jnp.tanh(encb * wh + hsum[i:i + 1, :])
        scT_s[:, i:i + 1] = lax.dot_general(
            a, vT, (((1,), (0,)), ((), ())),
            preferred_element_type=jnp.float32)
    scores = jnp.transpose(scT_s[...])
    m = jnp.max(scores, axis=1, keepdims=True)
    e = jnp.exp(scores - m)
    ad = e * (1.0 / jnp.sum(e, axis=1, keepdims=True))
    ad_ref[...] = ad
    adT = jnp.transpose(ad)
    for i in range(bn):
        encb = enc_ref[i]
        ctx_s[i:i + 1, :] = jnp.sum(encb * adT[:, i:i + 1], axis=0, keepdims=True)
    ctx = ctx_s[...]
    o = (lax.dot_general(h, ohwh_ref[...], (((1,), (1,)), ((), ())),
                         preferred_element_type=jnp.float32)
         + lax.dot_general(ctx, ohwc_ref[...], (((1,), (1,)), ((), ())),
                           preferred_element_type=jnp.float32)
         + ohb_ref[...])
    o_ref[...] = o
    g = (lax.dot_general(h, gwh_ref[...], (((1,), (1,)), ((), ())),
                         preferred_element_type=jnp.float32)
         + lax.dot_general(ctx, gwc_ref[...], (((1,), (1,)), ((), ())),
                           preferred_element_type=jnp.float32)
         + lax.dot_general(x_ref[...], gwx_ref[...], (((1,), (1,)), ((), ())),
                           preferred_element_type=jnp.float32)
         + genb_ref[0, 0])
    pgen_ref[...] = jnp.broadcast_to(jax.nn.sigmoid(g), pgen_ref.shape)
    qT = jnp.transpose(q_ref[...])
    lT = jnp.transpose(l_ref[...])
    iq = lax.broadcasted_iota(jnp.int32, (1, nrp), 1)
    il = lax.broadcasted_iota(jnp.int32, (1, 128), 1)
    for j in range(bn):
        pmat = jnp.where(qT[:, j:j + 1] == iq, adT[:, j:j + 1],
                         0.0).astype(jnp.bfloat16)
        momat = jnp.where(lT[:, j:j + 1] == il, 1.0, 0.0).astype(jnp.bfloat16)
        tb = lax.dot_general(pmat, momat, (((0,), (0,)), ((), ())),
                             preferred_element_type=jnp.float32)
        tid_ref[j * nrp:(j + 1) * nrp, :] = tb


def _b_kernel(o_ref, w_ref, b_ref, pg_ref, tid_ref, pv_ref, pf_ref,
              m_sc, s_sc, lg_sc, *, v_total, chunk, nc):
    step = pl.program_id(0)
    nb = o_ref.shape[0]

    @pl.when(step == 0)
    def _():
        m_sc[...] = jnp.full_like(m_sc, -3e38)
        s_sc[...] = jnp.zeros_like(s_sc)

    @pl.when(step < nc)
    def _():
        cg = step
        logits = lax.dot_general(o_ref[...], w_ref[...],
                                 (((1,), (1,)), ((), ())),
                                 preferred_element_type=jnp.float32) + b_ref[...]
        jcol = lax.broadcasted_iota(jnp.int32, logits.shape, 1)
        logits = jnp.where(jcol < (v_total - cg * chunk), logits, -1e30)
        for part in range(4):
            sl = slice(part * (chunk // 4), (part + 1) * (chunk // 4))
            lg_sc[cg, :, sl] = logits[:, sl]
        mc = jnp.max(logits, axis=1, keepdims=True)
        mn = jnp.maximum(m_sc[...], mc)
        s_sc[...] = (s_sc[...] * jnp.exp(m_sc[...] - mn)
                     + jnp.sum(jnp.exp(logits - mn), axis=1, keepdims=True))
        m_sc[...] = mn

    @pl.when(step >= nc)
    def _():
        cg = step - nc
        logits = lg_sc[cg]
        pv = jnp.exp(logits - m_sc[...]) * (1.0 / s_sc[...])
        pv_ref[...] = pv
        pg = pg_ref[:, 0:1]
        pf_ref[...] = pv * pg + (1.0 - pg) * tid_ref[...]


def kernel(input_token, last_decoder_hidden, encoder_states, full_input_var,
           emb_table, gru_w_ih, gru_w_hh, gru_b_ih, gru_b_hh,
           w_h, w_s, att_bias, attn_v, gen_w, gen_b,
           outh_w, outh_b, outv_w, outv_b):
    B, S, H = encoder_states.shape
    V, E = emb_table.shape
    PAD = 250
    VE = V + PAD
    NR = (VE + 127) // 128
    VEP = NR * 128
    CHUNK = 4096
    NC = 13
    BN = 16
    NBLK = (B // BN) // 2
    EB = B // 2

    f32 = jnp.float32
    cgc = lambda p, c: jnp.minimum(p * NCK + c, NC - 1)
    tok = input_token.reshape(B).astype(jnp.int32)
    emb3 = emb_table.reshape(V, 1, E)

    h_new, x = pl.pallas_call(
        _e_kernel,
        grid=(2, 1),
        in_specs=[
            pl.BlockSpec(memory_space=pltpu.SMEM),
            pl.BlockSpec(memory_space=pl.ANY),
            pl.BlockSpec((EB, H), lambda p, q: (p, 0)),
            pl.BlockSpec((3 * H, E), lambda p, q: (0, 0)),
            pl.BlockSpec((3 * H, H), lambda p, q: (0, 0)),
            pl.BlockSpec((1, 3 * H), lambda p, q: (0, 0)),
            pl.BlockSpec((1, 3 * H), lambda p, q: (0, 0)),
        ],
        out_specs=[
            pl.BlockSpec((EB, H), lambda p, q: (p, 0)),
            pl.BlockSpec((EB, E), lambda p, q: (p, 0)),
        ],
        out_shape=[
            jax.ShapeDtypeStruct((B, H), f32),
            jax.ShapeDtypeStruct((B, E), f32),
        ],
        scratch_shapes=[
            pltpu.VMEM((EB, 1, E), f32),
            pltpu.SemaphoreType.DMA,
        ],
        compiler_params=pltpu.CompilerParams(dimension_semantics=_SEM),
        name="embed_gru",
    )(tok, emb3, last_decoder_hidden, gru_w_ih, gru_w_hh,
      gru_b_ih.reshape(1, 3 * H), gru_b_hh.reshape(1, 3 * H))

    NRP = 400
    fiv = full_input_var.astype(jnp.int32)
    o, pgen_b, att_dist, tid = pl.pallas_call(
        functools.partial(_a_kernel, nrp=NRP),
        grid=(2, NBLK),
        in_specs=[
            pl.BlockSpec((BN, S, H), lambda p, i: (p * NBLK + i, 0, 0)),
            pl.BlockSpec((BN, H), lambda p, i: (p * NBLK + i, 0)),
            pl.BlockSpec((BN, E), lambda p, i: (p * NBLK + i, 0)),
            pl.BlockSpec((BN, S), lambda p, i: (p * NBLK + i, 0)),
            pl.BlockSpec((BN, S), lambda p, i: (p * NBLK + i, 0)),
            pl.BlockSpec((1, H), lambda p, i: (0, 0)),
            pl.BlockSpec((1, H), lambda p, i: (0, 0)),
            pl.BlockSpec((H, 1), lambda p, i: (0, 0)),
            pl.BlockSpec(memory_space=pltpu.SMEM),
            pl.BlockSpec((1, H), lambda p, i: (0, 0)),
            pl.BlockSpec((1, H), lambda p, i: (0, 0)),
            pl.BlockSpec((1, E), lambda p, i: (0, 0)),
            pl.BlockSpec(memory_space=pltpu.SMEM),
            pl.BlockSpec((E, H), lambda p, i: (0, 0)),
            pl.BlockSpec((E, H), lambda p, i: (0, 0)),
            pl.BlockSpec((1, E), lambda p, i: (0, 0)),
        ],
        out_specs=[
            pl.BlockSpec((BN, E), lambda p, i: (p * NBLK + i, 0)),
            pl.BlockSpec((BN, 128), lambda p, i: (p * NBLK + i, 0)),
            pl.BlockSpec((BN, S), lambda p, i: (p * NBLK + i, 0)),
            pl.BlockSpec((BN * NRP, 128), lambda p, i: (p * NBLK + i, 0)),
        ],
        out_shape=[
            jax.ShapeDtypeStruct((B, E), f32),
            jax.ShapeDtypeStruct((B, 128), f32),
            jax.ShapeDtypeStruct((B, S), f32),
            jax.ShapeDtypeStruct((B * NRP, 128), f32),
        ],
        scratch_shapes=[
            pltpu.VMEM((S, BN), f32),
            pltpu.VMEM((BN, H), f32),
        ],
        compiler_params=pltpu.CompilerParams(dimension_semantics=_SEM),
        name="attn_pgen",
    )(encoder_states, h_new, x, fiv >> 7, fiv & 127,
      w_h.reshape(1, H), w_s.reshape(1, H), attn_v.reshape(H, 1),
      att_bias.reshape(1, 1),
      gen_w[:, :H], gen_w[:, H:2 * H], gen_w[:, 2 * H:],
      gen_b.reshape(1, 1),
      outh_w[:, :H], outh_w[:, H:], outh_b.reshape(1, E))

    tid2 = tid.reshape(B, NRP * 128)

    def wb_idx(s):
        return jnp.where(s < NC, s, s - NC)

    def ph2_idx(s):
        return jnp.where(s < NC, 0, s - NC)

    p_vocab, p_final = pl.pallas_call(
        functools.partial(_b_kernel, v_total=V, chunk=CHUNK, nc=NC),
        grid=(2 * NC,),
        in_specs=[
            pl.BlockSpec((B, E), lambda s: (0, 0)),
            pl.BlockSpec((CHUNK, E), lambda s: (wb_idx(s), 0)),
            pl.BlockSpec((1, CHUNK), lambda s: (0, wb_idx(s))),
            pl.BlockSpec((B, 128), lambda s: (0, 0)),
            pl.BlockSpec((B, CHUNK), lambda s: (0, ph2_idx(s))),
        ],
        out_specs=[
            pl.BlockSpec((B, CHUNK), lambda s: (0, ph2_idx(s))),
            pl.BlockSpec((B, CHUNK), lambda s: (0, ph2_idx(s))),
        ],
        out_shape=[
            jax.ShapeDtypeStruct((B, V), f32),
            jax.ShapeDtypeStruct((B, VE), f32),
        ],
        scratch_shapes=[
            pltpu.VMEM((B, 1), f32),
            pltpu.VMEM((B, 1), f32),
            pltpu.VMEM((NC, B, CHUNK), f32),
        ],
        compiler_params=pltpu.CompilerParams(
            dimension_semantics=("arbitrary",),
            vmem_limit_bytes=58 * 1024 * 1024,
        ),
        name="vocab_softmax_mix",
    )(o, outv_w, outv_b.reshape(1, V), pgen_b, tid2)

    p_gen = pgen_b[:, 0:1]
    return (h_new, p_final, p_gen, p_vocab, att_dist)


# speedup vs baseline: 1.8180x; 1.0084x over previous
import functools

import jax
import jax.numpy as jnp
from jax import lax
from jax.experimental import pallas as pl
from jax.experimental.pallas import tpu as pltpu

_SEM = ("parallel", "arbitrary")


def _e_kernel(tok_ref, emb_ref, h_ref, wih_ref, whh_ref, bih_ref, bhh_ref,
              hnew_ref, x_ref, x3, sem):
    nb = x3.shape[0]
    base = pl.program_id(0) * nb
    for i in range(nb):
        pltpu.make_async_copy(emb_ref.at[tok_ref[base + i]], x3.at[i], sem).start()
    for i in range(nb):
        pltpu.make_async_copy(emb_ref.at[tok_ref[base + i]], x3.at[i], sem).wait()
    x = x3[...].reshape(nb, x3.shape[2])
    x_ref[...] = x
    h = h_ref[...]
    gi = lax.dot_general(x, wih_ref[...], (((1,), (1,)), ((), ())),
                         preferred_element_type=jnp.float32) + bih_ref[...]
    gh = lax.dot_general(h, whh_ref[...], (((1,), (1,)), ((), ())),
                         preferred_element_type=jnp.float32) + bhh_ref[...]
    hh = h.shape[1]
    r = jax.nn.sigmoid(gi[:, :hh] + gh[:, :hh])
    z = jax.nn.sigmoid(gi[:, hh:2 * hh] + gh[:, hh:2 * hh])
    n = jnp.tanh(gi[:, 2 * hh:] + r * gh[:, 2 * hh:])
    hnew_ref[...] = (1.0 - z) * n + z * h


def _a_kernel(enc_ref, h_ref, x_ref, q_ref, l_ref, wh_ref, ws_ref, vT_ref,
              attb_ref, gwh_ref, gwc_ref, gwx_ref, genb_ref, ohwh_ref,
              ohwc_ref, ohb_ref,
              o_ref, pgen_ref, ad_ref, tid_ref, scT_s, ctx_s, *, nrp):
    bn = h_ref.shape[0]
    h = h_ref[...]
    hsum = ws_ref[...] * h + attb_ref[0, 0]
    wh = wh_ref[...]
    vT = vT_ref[...]
    for i in range(bn):
        encb = enc_ref[i]
        a = jnp.tanh(encb * wh + hsum[i:i + 1, :])
        scT_s[:, i:i + 1] = lax.dot_general(
            a, vT, (((1,), (0,)), ((), ())),
            preferred_element_type=jnp.float32)
    scores = jnp.transpose(scT_s[...])
    m = jnp.max(scores, axis=1, keepdims=True)
    e = jnp.exp(scores - m)
    ad = e * (1.0 / jnp.sum(e, axis=1, keepdims=True))
    ad_ref[...] = ad
    adT = jnp.transpose(ad)
    for i in range(bn):
        encb = enc_ref[i]
        ctx_s[i:i + 1, :] = jnp.sum(encb * adT[:, i:i + 1], axis=0, keepdims=True)
    ctx = ctx_s[...]
    o = (lax.dot_general(h, ohwh_ref[...], (((1,), (1,)), ((), ())),
                         preferred_element_type=jnp.float32)
         + lax.dot_general(ctx, ohwc_ref[...], (((1,), (1,)), ((), ())),
                           preferred_element_type=jnp.float32)
         + ohb_ref[...])
    o_ref[...] = o
    g = (lax.dot_general(h, gwh_ref[...], (((1,), (1,)), ((), ())),
                         preferred_element_type=jnp.float32)
         + lax.dot_general(ctx, gwc_ref[...], (((1,), (1,)), ((), ())),
                           preferred_element_type=jnp.float32)
         + lax.dot_general(x_ref[...], gwx_ref[...], (((1,), (1,)), ((), ())),
                           preferred_element_type=jnp.float32)
         + genb_ref[0, 0])
    pgen_ref[...] = jnp.broadcast_to(jax.nn.sigmoid(g), pgen_ref.shape)
    qT = jnp.transpose(q_ref[...])
    lT = jnp.transpose(l_ref[...])
    iq = lax.broadcasted_iota(jnp.int32, (1, nrp), 1)
    il = lax.broadcasted_iota(jnp.int32, (1, 128), 1)
    for j in range(bn):
        pmat = jnp.where(qT[:, j:j + 1] == iq, adT[:, j:j + 1],
                         0.0).astype(jnp.bfloat16)
        momat = jnp.where(lT[:, j:j + 1] == il, 1.0, 0.0).astype(jnp.bfloat16)
        tb = lax.dot_general(pmat, momat, (((0,), (0,)), ((), ())),
                             preferred_element_type=jnp.float32)
        tid_ref[j * nrp:(j + 1) * nrp, :] = tb


def _b_kernel(o_ref, w_ref, b_ref, pg_ref, tid_ref, pv_ref, pf_ref,
              m_sc, s_sc, lg_sc, *, v_total, chunk, nc):
    step = pl.program_id(0)
    nb = o_ref.shape[0]

    @pl.when(step == 0)
    def _():
        m_sc[...] = jnp.full_like(m_sc, -3e38)
        s_sc[...] = jnp.zeros_like(s_sc)

    @pl.when(step < nc)
    def _():
        cg = step
        logits = lax.dot_general(o_ref[...], w_ref[...],
                                 (((1,), (1,)), ((), ())),
                                 preferred_element_type=jnp.float32) + b_ref[...]
        jcol = lax.broadcasted_iota(jnp.int32, logits.shape, 1)
        logits = jnp.where(jcol < (v_total - cg * chunk), logits, -1e30)
        for part in range(4):
            sl = slice(part * (chunk // 4), (part + 1) * (chunk // 4))
            lg_sc[cg, :, sl] = logits[:, sl]
        mc = jnp.max(logits, axis=1, keepdims=True)
        mn = jnp.maximum(m_sc[...], mc)
        s_sc[...] = (s_sc[...] * jnp.exp(m_sc[...] - mn)
                     + jnp.sum(jnp.exp(logits - mn), axis=1, keepdims=True))
        m_sc[...] = mn

    @pl.when(step >= nc)
    def _():
        cg = step - nc
        logits = lg_sc[cg]
        pv = jnp.exp(logits - m_sc[...]) * (1.0 / s_sc[...])
        pv_ref[...] = pv
        pg = pg_ref[:, 0:1]
        pf_ref[...] = pv * pg + (1.0 - pg) * tid_ref[...]


def kernel(input_token, last_decoder_hidden, encoder_states, full_input_var,
           emb_table, gru_w_ih, gru_w_hh, gru_b_ih, gru_b_hh,
           w_h, w_s, att_bias, attn_v, gen_w, gen_b,
           outh_w, outh_b, outv_w, outv_b):
    B, S, H = encoder_states.shape
    V, E = emb_table.shape
    PAD = 250
    VE = V + PAD
    NR = (VE + 127) // 128
    VEP = NR * 128
    CHUNK = 4096
    NC = 13
    BN = 32
    NBLK = (B // BN) // 2
    EB = B // 2

    f32 = jnp.float32
    cgc = lambda p, c: jnp.minimum(p * NCK + c, NC - 1)
    tok = input_token.reshape(B).astype(jnp.int32)
    emb3 = emb_table.reshape(V, 1, E)

    h_new, x = pl.pallas_call(
        _e_kernel,
        grid=(2, 1),
        in_specs=[
            pl.BlockSpec(memory_space=pltpu.SMEM),
            pl.BlockSpec(memory_space=pl.ANY),
            pl.BlockSpec((EB, H), lambda p, q: (p, 0)),
            pl.BlockSpec((3 * H, E), lambda p, q: (0, 0)),
            pl.BlockSpec((3 * H, H), lambda p, q: (0, 0)),
            pl.BlockSpec((1, 3 * H), lambda p, q: (0, 0)),
            pl.BlockSpec((1, 3 * H), lambda p, q: (0, 0)),
        ],
        out_specs=[
            pl.BlockSpec((EB, H), lambda p, q: (p, 0)),
            pl.BlockSpec((EB, E), lambda p, q: (p, 0)),
        ],
        out_shape=[
            jax.ShapeDtypeStruct((B, H), f32),
            jax.ShapeDtypeStruct((B, E), f32),
        ],
        scratch_shapes=[
            pltpu.VMEM((EB, 1, E), f32),
            pltpu.SemaphoreType.DMA,
        ],
        compiler_params=pltpu.CompilerParams(dimension_semantics=_SEM),
        name="embed_gru",
    )(tok, emb3, last_decoder_hidden, gru_w_ih, gru_w_hh,
      gru_b_ih.reshape(1, 3 * H), gru_b_hh.reshape(1, 3 * H))

    NRP = 400
    fiv = full_input_var.astype(jnp.int32)
    o, pgen_b, att_dist, tid = pl.pallas_call(
        functools.partial(_a_kernel, nrp=NRP),
        grid=(2, NBLK),
        in_specs=[
            pl.BlockSpec((BN, S, H), lambda p, i: (p * NBLK + i, 0, 0)),
            pl.BlockSpec((BN, H), lambda p, i: (p * NBLK + i, 0)),
            pl.BlockSpec((BN, E), lambda p, i: (p * NBLK + i, 0)),
            pl.BlockSpec((BN, S), lambda p, i: (p * NBLK + i, 0)),
            pl.BlockSpec((BN, S), lambda p, i: (p * NBLK + i, 0)),
            pl.BlockSpec((1, H), lambda p, i: (0, 0)),
            pl.BlockSpec((1, H), lambda p, i: (0, 0)),
            pl.BlockSpec((H, 1), lambda p, i: (0, 0)),
            pl.BlockSpec(memory_space=pltpu.SMEM),
            pl.BlockSpec((1, H), lambda p, i: (0, 0)),
            pl.BlockSpec((1, H), lambda p, i: (0, 0)),
            pl.BlockSpec((1, E), lambda p, i: (0, 0)),
            pl.BlockSpec(memory_space=pltpu.SMEM),
            pl.BlockSpec((E, H), lambda p, i: (0, 0)),
            pl.BlockSpec((E, H), lambda p, i: (0, 0)),
            pl.BlockSpec((1, E), lambda p, i: (0, 0)),
        ],
        out_specs=[
            pl.BlockSpec((BN, E), lambda p, i: (p * NBLK + i, 0)),
            pl.BlockSpec((BN, 128), lambda p, i: (p * NBLK + i, 0)),
            pl.BlockSpec((BN, S), lambda p, i: (p * NBLK + i, 0)),
            pl.BlockSpec((BN * NRP, 128), lambda p, i: (p * NBLK + i, 0)),
        ],
        out_shape=[
            jax.ShapeDtypeStruct((B, E), f32),
            jax.ShapeDtypeStruct((B, 128), f32),
            jax.ShapeDtypeStruct((B, S), f32),
            jax.ShapeDtypeStruct((B * NRP, 128), f32),
        ],
        scratch_shapes=[
            pltpu.VMEM((S, BN), f32),
            pltpu.VMEM((BN, H), f32),
        ],
        compiler_params=pltpu.CompilerParams(
            dimension_semantics=_SEM,
            vmem_limit_bytes=58 * 1024 * 1024,
        ),
        name="attn_pgen",
    )(encoder_states, h_new, x, fiv >> 7, fiv & 127,
      w_h.reshape(1, H), w_s.reshape(1, H), attn_v.reshape(H, 1),
      att_bias.reshape(1, 1),
      gen_w[:, :H], gen_w[:, H:2 * H], gen_w[:, 2 * H:],
      gen_b.reshape(1, 1),
      outh_w[:, :H], outh_w[:, H:], outh_b.reshape(1, E))

    tid2 = tid.reshape(B, NRP * 128)

    def wb_idx(s):
        return jnp.where(s < NC, s, s - NC)

    def ph2_idx(s):
        return jnp.where(s < NC, 0, s - NC)

    p_vocab, p_final = pl.pallas_call(
        functools.partial(_b_kernel, v_total=V, chunk=CHUNK, nc=NC),
        grid=(2 * NC,),
        in_specs=[
            pl.BlockSpec((B, E), lambda s: (0, 0)),
            pl.BlockSpec((CHUNK, E), lambda s: (wb_idx(s), 0)),
            pl.BlockSpec((1, CHUNK), lambda s: (0, wb_idx(s))),
            pl.BlockSpec((B, 128), lambda s: (0, 0)),
            pl.BlockSpec((B, CHUNK), lambda s: (0, ph2_idx(s))),
        ],
        out_specs=[
            pl.BlockSpec((B, CHUNK), lambda s: (0, ph2_idx(s))),
            pl.BlockSpec((B, CHUNK), lambda s: (0, ph2_idx(s))),
        ],
        out_shape=[
            jax.ShapeDtypeStruct((B, V), f32),
            jax.ShapeDtypeStruct((B, VE), f32),
        ],
        scratch_shapes=[
            pltpu.VMEM((B, 1), f32),
            pltpu.VMEM((B, 1), f32),
            pltpu.VMEM((NC, B, CHUNK), f32),
        ],
        compiler_params=pltpu.CompilerParams(
            dimension_semantics=("arbitrary",),
            vmem_limit_bytes=58 * 1024 * 1024,
        ),
        name="vocab_softmax_mix",
    )(o, outv_w, outv_b.reshape(1, V), pgen_b, tid2)

    p_gen = pgen_b[:, 0:1]
    return (h_new, p_final, p_gen, p_vocab, att_dist)
